# Initial kernel scaffold; baseline (speedup 1.0000x reference)
import jax
import jax.numpy as jnp
from jax.experimental import pallas as pl


def kernel(x, h, edge_index, edge_fea, v, em_W1, em_b1, em_W2, em_b2, cn_W1, cn_b1, cn_W2, cn_b2, nn_W1, nn_b1, nn_W2, nn_b2, nx_W1, nx_b1, nx_W2, nx_b2, nv_W1, nv_b1, nv_W2, nv_b2):
    raise NotImplementedError("write your pallas kernel here")


# speedup vs baseline: 1.5172x; 1.5172x over previous
import functools

import jax
import jax.numpy as jnp
from jax.experimental import pallas as pl
from jax.experimental.pallas import tpu as pltpu

H = 128


def _silu(u):
    return u * jax.nn.sigmoid(u)


def _pre_kernel(h_ref, w1r_ref, w1c_ref, nnw1t_ref,
                nxw1_ref, nxb1_ref, nxw2_ref,
                nvw1_ref, nvb1_ref, nvw2_ref,
                a_ref, b_ref, hpre_ref, g_ref):
    hh = h_ref[...]
    a_ref[...] = jnp.dot(hh, w1r_ref[...], preferred_element_type=jnp.float32)
    b_ref[...] = jnp.dot(hh, w1c_ref[...], preferred_element_type=jnp.float32)
    hpre_ref[...] = jnp.dot(hh, nnw1t_ref[...], preferred_element_type=jnp.float32)
    gx = _silu(jnp.dot(hh, nxw1_ref[...], preferred_element_type=jnp.float32)
               + nxb1_ref[...])
    gv = _silu(jnp.dot(hh, nvw1_ref[...], preferred_element_type=jnp.float32)
               + nvb1_ref[...])
    gxs = jnp.dot(gx, nxw2_ref[...], preferred_element_type=jnp.float32)
    gvs = jnp.dot(gv, nvw2_ref[...], preferred_element_type=jnp.float32)
    g_ref[...] = jnp.concatenate(
        [gxs, gvs, jnp.zeros_like(gxs), jnp.zeros_like(gvs)], axis=1)


def _precompute(h, w1r, w1c, nnw1t, nxw1, nxb1, nxw2, nvw1, nvb1, nvw2):
    n = h.shape[0]
    tn = 2000
    grid = (n // tn,)
    full = lambda shp: pl.BlockSpec(shp, lambda i: (0, 0))
    out = pl.pallas_call(
        _pre_kernel,
        grid=grid,
        in_specs=[
            pl.BlockSpec((tn, H), lambda i: (i, 0)),
            full((H, H)), full((H, H)), full((H, H)),
            full((H, H)), full((1, H)), full((H, 1)),
            full((H, H)), full((1, H)), full((H, 1)),
        ],
        out_specs=[
            pl.BlockSpec((tn, H), lambda i: (i, 0)),
            pl.BlockSpec((tn, H), lambda i: (i, 0)),
            pl.BlockSpec((tn, H), lambda i: (i, 0)),
            pl.BlockSpec((tn, 4), lambda i: (i, 0)),
        ],
        out_shape=[
            jax.ShapeDtypeStruct((n, H), jnp.float32),
            jax.ShapeDtypeStruct((n, H), jnp.float32),
            jax.ShapeDtypeStruct((n, H), jnp.float32),
            jax.ShapeDtypeStruct((n, 4), jnp.float32),
        ],
    )(h, w1r, w1c, nnw1t, nxw1, nxb1, nxw2, nvw1, nvb1, nvw2)
    return out


def _edge_kernel(g_ref, d_ref, ef_ref,
                 w1n_ref, w1ef_ref, b1_ref, w2_ref, b2_ref,
                 cnw1_ref, cnb1_ref, cnw2_ref, cnb2_ref,
                 msg_ref, fvec_ref):
    d = d_ref[...]
    rij = d[:, 0:3]
    vij = d[:, 3:6]
    nr = jnp.sqrt(jnp.sum(rij * rij, axis=1, keepdims=True))
    nv = jnp.sqrt(jnp.sum(vij * vij, axis=1, keepdims=True))
    w1n = w1n_ref[...]
    pre1 = (g_ref[...] + b1_ref[...]
            + nr * w1n[0:1, :] + nv * w1n[1:2, :]
            + jnp.dot(ef_ref[...], w1ef_ref[...],
                      preferred_element_type=jnp.float32))
    m1 = _silu(pre1)
    msg = _silu(jnp.dot(m1, w2_ref[...], preferred_element_type=jnp.float32)
                + b2_ref[...])
    msg_ref[...] = msg
    ch = _silu(jnp.dot(msg, cnw1_ref[...], preferred_element_type=jnp.float32)
               + cnb1_ref[...])
    cm = jnp.dot(ch, cnw2_ref[...], preferred_element_type=jnp.float32) \
        + cnb2_ref[0, 0]
    f = rij * cm
    ones = jnp.ones_like(cm)
    fvec_ref[...] = jnp.concatenate(
        [f, ones, jnp.zeros((f.shape[0], 12), jnp.float32)], axis=1)


def _edge_mlp(g, d, ef, w1n, w1ef, b1, w2, b2, cnw1, cnb1, cnw2, cnb2):
    e = g.shape[0]
    te = 2000
    grid = (e // te,)
    full = lambda shp: pl.BlockSpec(shp, lambda i: (0, 0))
    de = ef.shape[1]
    return pl.pallas_call(
        _edge_kernel,
        grid=grid,
        in_specs=[
            pl.BlockSpec((te, H), lambda i: (i, 0)),
            pl.BlockSpec((te, 16), lambda i: (i, 0)),
            pl.BlockSpec((te, de), lambda i: (i, 0)),
            full((2, H)), full((de, H)), full((1, H)),
            full((H, H)), full((1, H)),
            full((H, H)), full((1, H)), full((H, 1)), full((1, 1)),
        ],
        out_specs=[
            pl.BlockSpec((te, H), lambda i: (i, 0)),
            pl.BlockSpec((te, 16), lambda i: (i, 0)),
        ],
        out_shape=[
            jax.ShapeDtypeStruct((e, H), jnp.float32),
            jax.ShapeDtypeStruct((e, 16), jnp.float32),
        ],
    )(g, d, ef, w1n, w1ef, b1, w2, b2, cnw1, cnb1, cnw2, cnb2)


def _fin_kernel(hpre_ref, accm_ref, nnw1b_ref, nnb1_ref, nnw2_ref, nnb2_ref,
                hnew_ref):
    t = _silu(hpre_ref[...]
              + jnp.dot(accm_ref[...], nnw1b_ref[...],
                        preferred_element_type=jnp.float32)
              + nnb1_ref[...])
    hnew_ref[...] = jnp.dot(t, nnw2_ref[...],
                            preferred_element_type=jnp.float32) + nnb2_ref[...]


def _finalize(hpre, accm, nnw1b, nnb1, nnw2, nnb2):
    n = hpre.shape[0]
    tn = 2000
    full = lambda shp: pl.BlockSpec(shp, lambda i: (0, 0))
    return pl.pallas_call(
        _fin_kernel,
        grid=(n // tn,),
        in_specs=[
            pl.BlockSpec((tn, H), lambda i: (i, 0)),
            pl.BlockSpec((tn, H), lambda i: (i, 0)),
            full((H, H)), full((1, H)), full((H, H)), full((1, H)),
        ],
        out_specs=pl.BlockSpec((tn, H), lambda i: (i, 0)),
        out_shape=jax.ShapeDtypeStruct((n, H), jnp.float32),
    )(hpre, accm, nnw1b, nnb1, nnw2, nnb2)


def kernel(x, h, edge_index, edge_fea, v,
           em_W1, em_b1, em_W2, em_b2,
           cn_W1, cn_b1, cn_W2, cn_b2,
           nn_W1, nn_b1, nn_W2, nn_b2,
           nx_W1, nx_b1, nx_W2, nx_b2,
           nv_W1, nv_b1, nv_W2, nv_b2):
    n = x.shape[0]
    row = edge_index[0]
    col = edge_index[1]

    w1n = em_W1[0:2]
    w1r = em_W1[2:2 + H]
    w1c = em_W1[2 + H:2 + 2 * H]
    w1ef = em_W1[2 + 2 * H:]
    nnw1t = nn_W1[:H]
    nnw1b = nn_W1[H:]

    a, b, hpre, gates = _precompute(
        h, w1r, w1c, nnw1t,
        nx_W1, nx_b1.reshape(1, H), nx_W2,
        nv_W1, nv_b1.reshape(1, H), nv_W2)
    gate_x = gates[:, 0:1] + nx_b2[0]
    gate_v = gates[:, 1:2] + nv_b2[0]

    xv = jnp.concatenate([x, v, jnp.zeros((n, 10), jnp.float32)], axis=1)

    g = a[row] + b[col]
    d = xv[row] - xv[col]

    msg, fvec = _edge_mlp(
        g, d, edge_fea, w1n, w1ef,
        em_b1.reshape(1, H), em_W2, em_b2.reshape(1, H),
        cn_W1, cn_b1.reshape(1, H), cn_W2, cn_b2.reshape(1, 1))

    accm = jax.ops.segment_sum(msg, row, num_segments=n)
    accf = jax.ops.segment_sum(fvec, row, num_segments=n)

    cnt = accf[:, 3:4]
    tot_f = jnp.clip(accf[:, 0:3] / jnp.clip(cnt, 1.0, None), -100.0, 100.0)

    v_new = gate_v * v + tot_f
    x_new = gate_x * x + tot_f

    h_new = _finalize(hpre, accm, nnw1b,
                      nn_b1.reshape(1, H), nn_W2, nn_b2.reshape(1, H))
    return (x_new, v_new, h_new)


# speedup vs baseline: 1.9492x; 1.2848x over previous
import functools

import jax
import jax.numpy as jnp
from jax import lax
from jax.experimental import pallas as pl
from jax.experimental.pallas import tpu as pltpu
from jax.experimental.pallas import tpu_sc as plsc

H = 128
DD = 144
_NW = 32
_CH = 128


def _silu(u):
    return u * jax.nn.sigmoid(u)


def _pre_kernel(h_ref, w1r_ref, w1c_ref, nnw1t_ref,
                nxw1_ref, nxb1_ref, nxw2_ref,
                nvw1_ref, nvb1_ref, nvw2_ref,
                ta_ref, tb_ref, hpre_ref, g_ref):
    hh = h_ref[...]
    ta_ref[...] = jnp.dot(hh, w1r_ref[...], preferred_element_type=jnp.float32)
    tb_ref[...] = jnp.dot(hh, w1c_ref[...], preferred_element_type=jnp.float32)
    hpre_ref[...] = jnp.dot(hh, nnw1t_ref[...], preferred_element_type=jnp.float32)
    gx = _silu(jnp.dot(hh, nxw1_ref[...], preferred_element_type=jnp.float32)
               + nxb1_ref[...])
    gv = _silu(jnp.dot(hh, nvw1_ref[...], preferred_element_type=jnp.float32)
               + nvb1_ref[...])
    gxs = jnp.dot(gx, nxw2_ref[...], preferred_element_type=jnp.float32)
    gvs = jnp.dot(gv, nvw2_ref[...], preferred_element_type=jnp.float32)
    g_ref[...] = jnp.concatenate(
        [gxs, gvs, jnp.zeros_like(gxs), jnp.zeros_like(gvs)], axis=1)


def _precompute(h, w1r, w1c, nnw1t, nxw1, nxb1, nxw2, nvw1, nvb1, nvw2):
    n = h.shape[0]
    tn = 2000
    grid = (n // tn,)
    full = lambda shp: pl.BlockSpec(shp, lambda i: (0, 0))
    out = pl.pallas_call(
        _pre_kernel,
        grid=grid,
        in_specs=[
            pl.BlockSpec((tn, H), lambda i: (i, 0)),
            full((H, H)), full((H, H)), full((H, H)),
            full((H, H)), full((1, H)), full((H, 1)),
            full((H, H)), full((1, H)), full((H, 1)),
        ],
        out_specs=[
            pl.BlockSpec((tn, H), lambda i: (i, 0)),
            pl.BlockSpec((tn, H), lambda i: (i, 0)),
            pl.BlockSpec((tn, H), lambda i: (i, 0)),
            pl.BlockSpec((tn, 4), lambda i: (i, 0)),
        ],
        out_shape=[
            jax.ShapeDtypeStruct((n, H), jnp.float32),
            jax.ShapeDtypeStruct((n, H), jnp.float32),
            jax.ShapeDtypeStruct((n, H), jnp.float32),
            jax.ShapeDtypeStruct((n, 4), jnp.float32),
        ],
    )(h, w1r, w1c, nnw1t, nxw1, nxb1, nxw2, nvw1, nvb1, nvw2)
    return out


def _sc_gather(ta, tb, row, col):
    n, dd = ta.shape
    e = row.shape[0]
    nchunk = e // _CH
    nloop = (nchunk + _NW - 1) // _NW
    mesh = plsc.VectorSubcoreMesh(core_axis_name="c", subcore_axis_name="s")

    @functools.partial(
        pl.kernel,
        out_type=[jax.ShapeDtypeStruct((e, dd), jnp.float32),
                  jax.ShapeDtypeStruct((e, dd), jnp.float32)],
        mesh=mesh,
        scratch_types=[
            pltpu.VMEM((_CH,), jnp.int32),
            pltpu.VMEM((_CH,), jnp.int32),
            pltpu.VMEM((_CH, dd), jnp.float32),
            pltpu.VMEM((_CH, dd), jnp.float32),
            pltpu.SemaphoreType.DMA,
            pltpu.SemaphoreType.DMA,
        ],
    )
    def k(ta_hbm, tb_hbm, row_hbm, col_hbm, ga_hbm, gb_hbm,
          rowi, coli, abuf, bbuf, sema, semb):
        wid = lax.axis_index("s") * 2 + lax.axis_index("c")

        @pl.loop(0, nloop)
        def _(i):
            c = wid + i * _NW

            @pl.when(c < nchunk)
            def _():
                base = c * _CH
                pltpu.sync_copy(row_hbm.at[pl.ds(base, _CH)], rowi)
                pltpu.sync_copy(col_hbm.at[pl.ds(base, _CH)], coli)
                ca = pltpu.async_copy(ta_hbm.at[rowi], abuf, sema)
                cb = pltpu.async_copy(tb_hbm.at[coli], bbuf, semb)
                ca.wait()
                cb.wait()
                pltpu.sync_copy(abuf, ga_hbm.at[pl.ds(base, _CH)])
                pltpu.sync_copy(bbuf, gb_hbm.at[pl.ds(base, _CH)])

    return k(ta, tb, row, col)


def _edge_kernel(ga_ref, gb_ref, d_ref, ef_ref,
                 w1n_ref, w1ef_ref, b1_ref, w2_ref, b2_ref,
                 cnw1_ref, cnb1_ref, cnw2_ref, cnb2_ref,
                 msg_ref, fvec_ref):
    ga = ga_ref[...]
    gb = gb_ref[...]
    d = d_ref[...]
    rij = d[:, 0:3]
    vij = d[:, 3:6]
    nr = jnp.sqrt(jnp.sum(rij * rij, axis=1, keepdims=True))
    nv = jnp.sqrt(jnp.sum(vij * vij, axis=1, keepdims=True))
    w1n = w1n_ref[...]
    pre1 = (ga + gb + b1_ref[...]
            + nr * w1n[0:1, :] + nv * w1n[1:2, :]
            + jnp.dot(ef_ref[...], w1ef_ref[...],
                      preferred_element_type=jnp.float32))
    m1 = _silu(pre1)
    msg = _silu(jnp.dot(m1, w2_ref[...], preferred_element_type=jnp.float32)
                + b2_ref[...])
    msg_ref[...] = msg
    ch = _silu(jnp.dot(msg, cnw1_ref[...], preferred_element_type=jnp.float32)
               + cnb1_ref[...])
    cm = jnp.dot(ch, cnw2_ref[...], preferred_element_type=jnp.float32) \
        + cnb2_ref[0, 0]
    f = rij * cm
    ones = jnp.ones_like(cm)
    fvec_ref[...] = jnp.concatenate(
        [f, ones, jnp.zeros((f.shape[0], 12), jnp.float32)], axis=1)


def _edge_mlp(ga, gb, d, ef, w1n, w1ef, b1, w2, b2, cnw1, cnb1, cnw2, cnb2):
    e = ga.shape[0]
    te = 2000
    grid = (e // te,)
    full = lambda shp: pl.BlockSpec(shp, lambda i: (0, 0))
    de = ef.shape[1]
    return pl.pallas_call(
        _edge_kernel,
        grid=grid,
        in_specs=[
            pl.BlockSpec((te, H), lambda i: (i, 0)),
            pl.BlockSpec((te, H), lambda i: (i, 0)),
            pl.BlockSpec((te, 8), lambda i: (i, 0)),
            pl.BlockSpec((te, de), lambda i: (i, 0)),
            full((2, H)), full((de, H)), full((1, H)),
            full((H, H)), full((1, H)),
            full((H, H)), full((1, H)), full((H, 1)), full((1, 1)),
        ],
        out_specs=[
            pl.BlockSpec((te, H), lambda i: (i, 0)),
            pl.BlockSpec((te, 16), lambda i: (i, 0)),
        ],
        out_shape=[
            jax.ShapeDtypeStruct((e, H), jnp.float32),
            jax.ShapeDtypeStruct((e, 16), jnp.float32),
        ],
    )(ga, gb, d, ef, w1n, w1ef, b1, w2, b2, cnw1, cnb1, cnw2, cnb2)


def _fin_kernel(hpre_ref, accm_ref, nnw1b_ref, nnb1_ref, nnw2_ref, nnb2_ref,
                hnew_ref):
    t = _silu(hpre_ref[...]
              + jnp.dot(accm_ref[...], nnw1b_ref[...],
                        preferred_element_type=jnp.float32)
              + nnb1_ref[...])
    hnew_ref[...] = jnp.dot(t, nnw2_ref[...],
                            preferred_element_type=jnp.float32) + nnb2_ref[...]


def _finalize(hpre, accm, nnw1b, nnb1, nnw2, nnb2):
    n = hpre.shape[0]
    tn = 2000
    full = lambda shp: pl.BlockSpec(shp, lambda i: (0, 0))
    return pl.pallas_call(
        _fin_kernel,
        grid=(n // tn,),
        in_specs=[
            pl.BlockSpec((tn, H), lambda i: (i, 0)),
            pl.BlockSpec((tn, H), lambda i: (i, 0)),
            full((H, H)), full((1, H)), full((H, H)), full((1, H)),
        ],
        out_specs=pl.BlockSpec((tn, H), lambda i: (i, 0)),
        out_shape=jax.ShapeDtypeStruct((n, H), jnp.float32),
    )(hpre, accm, nnw1b, nnb1, nnw2, nnb2)


def kernel(x, h, edge_index, edge_fea, v,
           em_W1, em_b1, em_W2, em_b2,
           cn_W1, cn_b1, cn_W2, cn_b2,
           nn_W1, nn_b1, nn_W2, nn_b2,
           nx_W1, nx_b1, nx_W2, nx_b2,
           nv_W1, nv_b1, nv_W2, nv_b2):
    n = x.shape[0]
    row = edge_index[0]
    col = edge_index[1]

    w1n = em_W1[0:2]
    w1r = em_W1[2:2 + H]
    w1c = em_W1[2 + H:2 + 2 * H]
    w1ef = em_W1[2 + 2 * H:]
    nnw1t = nn_W1[:H]
    nnw1b = nn_W1[H:]

    ta, tb, hpre, gates = _precompute(
        h, w1r, w1c, nnw1t,
        nx_W1, nx_b1.reshape(1, H), nx_W2,
        nv_W1, nv_b1.reshape(1, H), nv_W2)
    gate_x = gates[:, 0:1] + nx_b2[0]
    gate_v = gates[:, 1:2] + nv_b2[0]

    ga, gb = _sc_gather(ta, tb, row, col)

    xv = jnp.concatenate([x, v, jnp.zeros((n, 2), jnp.float32)], axis=1)
    d = xv[row] - xv[col]

    msg, fvec = _edge_mlp(
        ga, gb, d, edge_fea, w1n, w1ef,
        em_b1.reshape(1, H), em_W2, em_b2.reshape(1, H),
        cn_W1, cn_b1.reshape(1, H), cn_W2, cn_b2.reshape(1, 1))

    accm = jax.ops.segment_sum(msg, row, num_segments=n)
    accf = jax.ops.segment_sum(fvec, row, num_segments=n)

    cnt = accf[:, 3:4]
    tot_f = jnp.clip(accf[:, 0:3] / jnp.clip(cnt, 1.0, None), -100.0, 100.0)

    v_new = gate_v * v + tot_f
    x_new = gate_x * x + tot_f

    h_new = _finalize(hpre, accm, nnw1b,
                      nn_b1.reshape(1, H), nn_W2, nn_b2.reshape(1, H))
    return (x_new, v_new, h_new)


# speedup vs baseline: 2.2447x; 1.1516x over previous
import functools

import jax
import jax.numpy as jnp
from jax import lax
from jax.experimental import pallas as pl
from jax.experimental.pallas import tpu as pltpu
from jax.experimental.pallas import tpu_sc as plsc

H = 128
DD = 144
_NW = 32
_CH = 128


def _silu(u):
    return u * jax.nn.sigmoid(u)


def _pre_kernel(h_ref, w1r_ref, w1c_ref, nnw1t_ref,
                nxw1_ref, nxb1_ref, nxw2_ref,
                nvw1_ref, nvb1_ref, nvw2_ref,
                ta_ref, tb_ref, hpre_ref, g_ref):
    hh = h_ref[...]
    ta_ref[...] = jnp.dot(hh, w1r_ref[...], preferred_element_type=jnp.float32)
    tb_ref[...] = jnp.dot(hh, w1c_ref[...], preferred_element_type=jnp.float32)
    hpre_ref[...] = jnp.dot(hh, nnw1t_ref[...], preferred_element_type=jnp.float32)
    gx = _silu(jnp.dot(hh, nxw1_ref[...], preferred_element_type=jnp.float32)
               + nxb1_ref[...])
    gv = _silu(jnp.dot(hh, nvw1_ref[...], preferred_element_type=jnp.float32)
               + nvb1_ref[...])
    gxs = jnp.dot(gx, nxw2_ref[...], preferred_element_type=jnp.float32)
    gvs = jnp.dot(gv, nvw2_ref[...], preferred_element_type=jnp.float32)
    g_ref[...] = jnp.concatenate(
        [gxs, gvs, jnp.zeros_like(gxs), jnp.zeros_like(gvs)], axis=1)


def _precompute(h, w1r, w1c, nnw1t, nxw1, nxb1, nxw2, nvw1, nvb1, nvw2):
    n = h.shape[0]
    tn = 2000
    grid = (n // tn,)
    full = lambda shp: pl.BlockSpec(shp, lambda i: (0, 0))
    out = pl.pallas_call(
        _pre_kernel,
        grid=grid,
        in_specs=[
            pl.BlockSpec((tn, H), lambda i: (i, 0)),
            full((H, H)), full((H, H)), full((H, H)),
            full((H, H)), full((1, H)), full((H, 1)),
            full((H, H)), full((1, H)), full((H, 1)),
        ],
        out_specs=[
            pl.BlockSpec((tn, H), lambda i: (i, 0)),
            pl.BlockSpec((tn, H), lambda i: (i, 0)),
            pl.BlockSpec((tn, H), lambda i: (i, 0)),
            pl.BlockSpec((tn, 4), lambda i: (i, 0)),
        ],
        out_shape=[
            jax.ShapeDtypeStruct((n, H), jnp.float32),
            jax.ShapeDtypeStruct((n, H), jnp.float32),
            jax.ShapeDtypeStruct((n, H), jnp.float32),
            jax.ShapeDtypeStruct((n, 4), jnp.float32),
        ],
    )(h, w1r, w1c, nnw1t, nxw1, nxb1, nxw2, nvw1, nvb1, nvw2)
    return out


def _sc_gather(ta, tb, row, col):
    n, dd = ta.shape
    e = row.shape[0]
    nchunk = e // _CH
    nloop = (nchunk + _NW - 1) // _NW
    mesh = plsc.VectorSubcoreMesh(core_axis_name="c", subcore_axis_name="s")

    @functools.partial(
        pl.kernel,
        out_type=[jax.ShapeDtypeStruct((e, dd), jnp.float32),
                  jax.ShapeDtypeStruct((e, dd), jnp.float32)],
        mesh=mesh,
        scratch_types=[
            pltpu.VMEM((_CH,), jnp.int32),
            pltpu.VMEM((_CH,), jnp.int32),
            pltpu.VMEM((_CH, dd), jnp.float32),
            pltpu.VMEM((_CH, dd), jnp.float32),
            pltpu.SemaphoreType.DMA,
            pltpu.SemaphoreType.DMA,
        ],
    )
    def k(ta_hbm, tb_hbm, row_hbm, col_hbm, ga_hbm, gb_hbm,
          rowi, coli, abuf, bbuf, sema, semb):
        wid = lax.axis_index("s") * 2 + lax.axis_index("c")

        @pl.loop(0, nloop)
        def _(i):
            c = wid + i * _NW

            @pl.when(c < nchunk)
            def _():
                base = c * _CH
                pltpu.sync_copy(row_hbm.at[pl.ds(base, _CH)], rowi)
                pltpu.sync_copy(col_hbm.at[pl.ds(base, _CH)], coli)
                ca = pltpu.async_copy(ta_hbm.at[rowi], abuf, sema)
                cb = pltpu.async_copy(tb_hbm.at[coli], bbuf, semb)
                ca.wait()
                cb.wait()
                pltpu.sync_copy(abuf, ga_hbm.at[pl.ds(base, _CH)])
                pltpu.sync_copy(bbuf, gb_hbm.at[pl.ds(base, _CH)])

    return k(ta, tb, row, col)


def _sc_scatter(msg, row, zm):
    e, hh = msg.shape
    n = zm.shape[0]
    nchunk = e // _CH
    half = nchunk // 2
    nsub = 16
    nloop = (half + nsub - 1) // nsub
    stripe = n // nsub
    mesh = plsc.VectorSubcoreMesh(core_axis_name="c", subcore_axis_name="s")

    @functools.partial(
        pl.kernel,
        out_type=jax.ShapeDtypeStruct((2, n, hh), jnp.float32),
        mesh=mesh,
        scratch_types=[
            pltpu.VMEM((_CH,), jnp.int32),
            pltpu.VMEM((_CH, hh), jnp.float32),
            pltpu.VMEM_SHARED((n, hh), jnp.float32),
        ],
    )
    def k(msg_hbm, row_hbm, zm_hbm, outm_hbm, rowi, mbuf, accm):
        c = lax.axis_index("c")
        s = lax.axis_index("s")
        sz_a, sz_b = 624, n - 15 * 624

        def _zero(r0, sz):
            pltpu.sync_copy(zm_hbm.at[pl.ds(r0, sz)], accm.at[pl.ds(r0, sz)])

        @pl.when(s < 15)
        def _():
            _zero(pl.multiple_of(s * sz_a, 8), sz_a)

        @pl.when(s == 15)
        def _():
            _zero(15 * sz_a, sz_b)

        plsc.subcore_barrier()

        @pl.loop(0, nloop)
        def _(i):
            j = s + i * nsub

            @pl.when(j < half)
            def _():
                base = (c * half + j) * _CH
                pltpu.sync_copy(row_hbm.at[pl.ds(base, _CH)], rowi)
                pltpu.sync_copy(msg_hbm.at[pl.ds(base, _CH)], mbuf)
                pltpu.sync_copy(mbuf, accm.at[rowi], add=True)

        plsc.subcore_barrier()

        def _wb(r0, sz):
            pltpu.sync_copy(accm.at[pl.ds(r0, sz)],
                            outm_hbm.at[c, pl.ds(r0, sz)])

        @pl.when(s < 15)
        def _():
            _wb(pl.multiple_of(s * sz_a, 8), sz_a)

        @pl.when(s == 15)
        def _():
            _wb(15 * sz_a, sz_b)

    return k(msg, row, zm)


def _edge_kernel(ga_ref, gb_ref, d_ref, ef_ref,
                 w1n_ref, w1ef_ref, b1_ref, w2_ref, b2_ref,
                 cnw1_ref, cnb1_ref, cnw2_ref, cnb2_ref,
                 msg_ref, fvec_ref):
    ga = ga_ref[...]
    gb = gb_ref[...]
    d = d_ref[...]
    rij = d[:, 0:3]
    vij = d[:, 3:6]
    nr = jnp.sqrt(jnp.sum(rij * rij, axis=1, keepdims=True))
    nv = jnp.sqrt(jnp.sum(vij * vij, axis=1, keepdims=True))
    w1n = w1n_ref[...]
    pre1 = (ga + gb + b1_ref[...]
            + nr * w1n[0:1, :] + nv * w1n[1:2, :]
            + jnp.dot(ef_ref[...], w1ef_ref[...],
                      preferred_element_type=jnp.float32))
    m1 = _silu(pre1)
    msg = _silu(jnp.dot(m1, w2_ref[...], preferred_element_type=jnp.float32)
                + b2_ref[...])
    msg_ref[...] = msg
    ch = _silu(jnp.dot(msg, cnw1_ref[...], preferred_element_type=jnp.float32)
               + cnb1_ref[...])
    cm = jnp.dot(ch, cnw2_ref[...], preferred_element_type=jnp.float32) \
        + cnb2_ref[0, 0]
    f = rij * cm
    ones = jnp.ones_like(cm)
    fvec_ref[...] = jnp.concatenate(
        [f, ones, jnp.zeros((f.shape[0], 12), jnp.float32)], axis=1)


def _edge_mlp(ga, gb, d, ef, w1n, w1ef, b1, w2, b2, cnw1, cnb1, cnw2, cnb2):
    e = ga.shape[0]
    te = 2000
    grid = (e // te,)
    full = lambda shp: pl.BlockSpec(shp, lambda i: (0, 0))
    de = ef.shape[1]
    return pl.pallas_call(
        _edge_kernel,
        grid=grid,
        in_specs=[
            pl.BlockSpec((te, H), lambda i: (i, 0)),
            pl.BlockSpec((te, H), lambda i: (i, 0)),
            pl.BlockSpec((te, 8), lambda i: (i, 0)),
            pl.BlockSpec((te, de), lambda i: (i, 0)),
            full((2, H)), full((de, H)), full((1, H)),
            full((H, H)), full((1, H)),
            full((H, H)), full((1, H)), full((H, 1)), full((1, 1)),
        ],
        out_specs=[
            pl.BlockSpec((te, H), lambda i: (i, 0)),
            pl.BlockSpec((te, 16), lambda i: (i, 0)),
        ],
        out_shape=[
            jax.ShapeDtypeStruct((e, H), jnp.float32),
            jax.ShapeDtypeStruct((e, 16), jnp.float32),
        ],
    )(ga, gb, d, ef, w1n, w1ef, b1, w2, b2, cnw1, cnb1, cnw2, cnb2)


def _fin_kernel(hpre_ref, accm0_ref, accm1_ref,
                nnw1b_ref, nnb1_ref, nnw2_ref, nnb2_ref, hnew_ref):
    accm = accm0_ref[0] + accm1_ref[0]
    t = _silu(hpre_ref[...]
              + jnp.dot(accm, nnw1b_ref[...],
                        preferred_element_type=jnp.float32)
              + nnb1_ref[...])
    hnew_ref[...] = jnp.dot(t, nnw2_ref[...],
                            preferred_element_type=jnp.float32) + nnb2_ref[...]


def _finalize(hpre, accm2, nnw1b, nnb1, nnw2, nnb2):
    n = hpre.shape[0]
    tn = 2000
    full = lambda shp: pl.BlockSpec(shp, lambda i: (0, 0))
    return pl.pallas_call(
        _fin_kernel,
        grid=(n // tn,),
        in_specs=[
            pl.BlockSpec((tn, H), lambda i: (i, 0)),
            pl.BlockSpec((1, tn, H), lambda i: (0, i, 0)),
            pl.BlockSpec((1, tn, H), lambda i: (1, i, 0)),
            full((H, H)), full((1, H)), full((H, H)), full((1, H)),
        ],
        out_specs=pl.BlockSpec((tn, H), lambda i: (i, 0)),
        out_shape=jax.ShapeDtypeStruct((n, H), jnp.float32),
    )(hpre, accm2, accm2, nnw1b, nnb1, nnw2, nnb2)


def kernel(x, h, edge_index, edge_fea, v,
           em_W1, em_b1, em_W2, em_b2,
           cn_W1, cn_b1, cn_W2, cn_b2,
           nn_W1, nn_b1, nn_W2, nn_b2,
           nx_W1, nx_b1, nx_W2, nx_b2,
           nv_W1, nv_b1, nv_W2, nv_b2):
    n = x.shape[0]
    row = edge_index[0]
    col = edge_index[1]

    w1n = em_W1[0:2]
    w1r = em_W1[2:2 + H]
    w1c = em_W1[2 + H:2 + 2 * H]
    w1ef = em_W1[2 + 2 * H:]
    nnw1t = nn_W1[:H]
    nnw1b = nn_W1[H:]

    ta, tb, hpre, gates = _precompute(
        h, w1r, w1c, nnw1t,
        nx_W1, nx_b1.reshape(1, H), nx_W2,
        nv_W1, nv_b1.reshape(1, H), nv_W2)
    gate_x = gates[:, 0:1] + nx_b2[0]
    gate_v = gates[:, 1:2] + nv_b2[0]

    ga, gb = _sc_gather(ta, tb, row, col)

    xv = jnp.concatenate([x, v, jnp.zeros((n, 2), jnp.float32)], axis=1)
    d = xv[row] - xv[col]

    msg, fvec = _edge_mlp(
        ga, gb, d, edge_fea, w1n, w1ef,
        em_b1.reshape(1, H), em_W2, em_b2.reshape(1, H),
        cn_W1, cn_b1.reshape(1, H), cn_W2, cn_b2.reshape(1, 1))

    zm = jnp.zeros((n, H), jnp.float32)
    accm2 = _sc_scatter(msg, row, zm)
    accf = jax.ops.segment_sum(fvec, row, num_segments=n)

    cnt = accf[:, 3:4]
    tot_f = jnp.clip(accf[:, 0:3] / jnp.clip(cnt, 1.0, None), -100.0, 100.0)

    v_new = gate_v * v + tot_f
    x_new = gate_x * x + tot_f

    h_new = _finalize(hpre, accm2, nnw1b,
                      nn_b1.reshape(1, H), nn_W2, nn_b2.reshape(1, H))
    return (x_new, v_new, h_new)


# speedup vs baseline: 3.8982x; 1.7367x over previous
import functools

import jax
import jax.numpy as jnp
from jax import lax
from jax.experimental import pallas as pl
from jax.experimental.pallas import tpu as pltpu
from jax.experimental.pallas import tpu_sc as plsc

H = 128
DD = 144
_NW = 32
_CH = 128


def _silu(u):
    return u * jax.nn.sigmoid(u)


def _pre_kernel(h_ref, w1r_ref, w1c_ref, nnw1t_ref,
                nxw1_ref, nxb1_ref, nxw2_ref,
                nvw1_ref, nvb1_ref, nvw2_ref,
                ta_ref, tb_ref, hpre_ref, g_ref):
    hh = h_ref[...]
    ta_ref[...] = jnp.dot(hh, w1r_ref[...], preferred_element_type=jnp.float32)
    tb_ref[...] = jnp.dot(hh, w1c_ref[...], preferred_element_type=jnp.float32)
    hpre_ref[...] = jnp.dot(hh, nnw1t_ref[...], preferred_element_type=jnp.float32)
    gx = _silu(jnp.dot(hh, nxw1_ref[...], preferred_element_type=jnp.float32)
               + nxb1_ref[...])
    gv = _silu(jnp.dot(hh, nvw1_ref[...], preferred_element_type=jnp.float32)
               + nvb1_ref[...])
    gxs = jnp.dot(gx, nxw2_ref[...], preferred_element_type=jnp.float32)
    gvs = jnp.dot(gv, nvw2_ref[...], preferred_element_type=jnp.float32)
    g_ref[...] = jnp.concatenate(
        [gxs, gvs, jnp.zeros_like(gxs), jnp.zeros_like(gvs)], axis=1)


def _precompute(h, w1r, w1c, nnw1t, nxw1, nxb1, nxw2, nvw1, nvb1, nvw2):
    n = h.shape[0]
    tn = 2000
    grid = (n // tn,)
    full = lambda shp: pl.BlockSpec(shp, lambda i: (0, 0))
    out = pl.pallas_call(
        _pre_kernel,
        grid=grid,
        in_specs=[
            pl.BlockSpec((tn, H), lambda i: (i, 0)),
            full((H, H)), full((H, H)), full((H, H)),
            full((H, H)), full((1, H)), full((H, 1)),
            full((H, H)), full((1, H)), full((H, 1)),
        ],
        out_specs=[
            pl.BlockSpec((tn, H), lambda i: (i, 0)),
            pl.BlockSpec((tn, H), lambda i: (i, 0)),
            pl.BlockSpec((tn, H), lambda i: (i, 0)),
            pl.BlockSpec((tn, 4), lambda i: (i, 0)),
        ],
        out_shape=[
            jax.ShapeDtypeStruct((n, H), jnp.float32),
            jax.ShapeDtypeStruct((n, H), jnp.float32),
            jax.ShapeDtypeStruct((n, H), jnp.float32),
            jax.ShapeDtypeStruct((n, 4), jnp.float32),
        ],
    )(h, w1r, w1c, nnw1t, nxw1, nxb1, nxw2, nvw1, nvb1, nvw2)
    return out


def _sc_gather(ta, tb, xv, row, col):
    n, dd = ta.shape
    e = row.shape[0]
    nchunk = e // _CH
    nloop = (nchunk + _NW - 1) // _NW
    mesh = plsc.VectorSubcoreMesh(core_axis_name="c", subcore_axis_name="s")

    @functools.partial(
        pl.kernel,
        out_type=[jax.ShapeDtypeStruct((e, dd), jnp.float32),
                  jax.ShapeDtypeStruct((e, dd), jnp.float32),
                  jax.ShapeDtypeStruct((e, 16), jnp.float32)],
        mesh=mesh,
        scratch_types=[
            pltpu.VMEM((_CH,), jnp.int32),
            pltpu.VMEM((_CH,), jnp.int32),
            pltpu.VMEM((_CH, dd), jnp.float32),
            pltpu.VMEM((_CH, dd), jnp.float32),
            pltpu.VMEM((_CH, dd), jnp.float32),
            pltpu.VMEM((_CH, dd), jnp.float32),
            pltpu.VMEM((_CH, 16), jnp.float32),
            pltpu.SemaphoreType.DMA,
            pltpu.SemaphoreType.DMA,
            pltpu.SemaphoreType.DMA,
            pltpu.SemaphoreType.DMA,
        ],
    )
    def k(ta_hbm, tb_hbm, xv_hbm, row_hbm, col_hbm, ga_hbm, gb_hbm, d_hbm,
          rowi, coli, abuf, bbuf, xr, xc, dbuf, sema, semb, semc, semd):
        wid = lax.axis_index("s") * 2 + lax.axis_index("c")

        @pl.loop(0, nloop)
        def _(i):
            c = wid + i * _NW

            @pl.when(c < nchunk)
            def _():
                base = c * _CH
                pltpu.sync_copy(row_hbm.at[pl.ds(base, _CH)], rowi)
                pltpu.sync_copy(col_hbm.at[pl.ds(base, _CH)], coli)
                ca = pltpu.async_copy(ta_hbm.at[rowi], abuf, sema)
                cb = pltpu.async_copy(tb_hbm.at[coli], bbuf, semb)
                cr = pltpu.async_copy(xv_hbm.at[rowi], xr, semc)
                cc = pltpu.async_copy(xv_hbm.at[coli], xc, semd)
                cr.wait()
                cc.wait()
                for r in range(_CH):
                    dbuf[r, :] = xr[r, pl.ds(0, 16)] - xc[r, pl.ds(0, 16)]
                ca.wait()
                cb.wait()
                pltpu.sync_copy(abuf, ga_hbm.at[pl.ds(base, _CH)])
                pltpu.sync_copy(bbuf, gb_hbm.at[pl.ds(base, _CH)])
                pltpu.sync_copy(dbuf, d_hbm.at[pl.ds(base, _CH)])

    return k(ta, tb, xv, row, col)


def _sc_scatter(msg, row, zm):
    e, hh = msg.shape
    n = zm.shape[0]
    nchunk = e // _CH
    half = nchunk // 2
    nsub = 16
    nloop = (half + nsub - 1) // nsub
    stripe = n // nsub
    mesh = plsc.VectorSubcoreMesh(core_axis_name="c", subcore_axis_name="s")

    @functools.partial(
        pl.kernel,
        out_type=jax.ShapeDtypeStruct((2, n, hh), jnp.float32),
        mesh=mesh,
        scratch_types=[
            pltpu.VMEM((_CH,), jnp.int32),
            pltpu.VMEM((_CH, hh), jnp.float32),
            pltpu.VMEM_SHARED((n, hh), jnp.float32),
        ],
    )
    def k(msg_hbm, row_hbm, zm_hbm, outm_hbm, rowi, mbuf, accm):
        c = lax.axis_index("c")
        s = lax.axis_index("s")
        sz_a, sz_b = 624, n - 15 * 624

        def _zero(r0, sz):
            pltpu.sync_copy(zm_hbm.at[pl.ds(r0, sz)], accm.at[pl.ds(r0, sz)])

        @pl.when(s < 15)
        def _():
            _zero(pl.multiple_of(s * sz_a, 8), sz_a)

        @pl.when(s == 15)
        def _():
            _zero(15 * sz_a, sz_b)

        plsc.subcore_barrier()

        @pl.loop(0, nloop)
        def _(i):
            j = s + i * nsub

            @pl.when(j < half)
            def _():
                base = (c * half + j) * _CH
                pltpu.sync_copy(row_hbm.at[pl.ds(base, _CH)], rowi)
                pltpu.sync_copy(msg_hbm.at[pl.ds(base, _CH)], mbuf)
                pltpu.sync_copy(mbuf, accm.at[rowi], add=True)

        plsc.subcore_barrier()

        def _wb(r0, sz):
            pltpu.sync_copy(accm.at[pl.ds(r0, sz)],
                            outm_hbm.at[c, pl.ds(r0, sz)])

        @pl.when(s < 15)
        def _():
            _wb(pl.multiple_of(s * sz_a, 8), sz_a)

        @pl.when(s == 15)
        def _():
            _wb(15 * sz_a, sz_b)

    return k(msg, row, zm)


def _edge_kernel(ga_ref, gb_ref, d_ref, ef_ref,
                 w1n_ref, w1ef_ref, b1_ref, w2_ref, b2_ref,
                 cnw1_ref, cnb1_ref, cnw2_ref, cnb2_ref,
                 msg_ref, fvec_ref):
    ga = ga_ref[...]
    gb = gb_ref[...]
    d = d_ref[...]
    rij = d[:, 0:3]
    vij = d[:, 3:6]
    nr = jnp.sqrt(jnp.sum(rij * rij, axis=1, keepdims=True))
    nv = jnp.sqrt(jnp.sum(vij * vij, axis=1, keepdims=True))
    w1n = w1n_ref[...]
    pre1 = (ga + gb + b1_ref[...]
            + nr * w1n[0:1, :] + nv * w1n[1:2, :]
            + jnp.dot(ef_ref[...], w1ef_ref[...],
                      preferred_element_type=jnp.float32))
    m1 = _silu(pre1)
    msg = _silu(jnp.dot(m1, w2_ref[...], preferred_element_type=jnp.float32)
                + b2_ref[...])
    msg_ref[...] = msg
    ch = _silu(jnp.dot(msg, cnw1_ref[...], preferred_element_type=jnp.float32)
               + cnb1_ref[...])
    cm = jnp.dot(ch, cnw2_ref[...], preferred_element_type=jnp.float32) \
        + cnb2_ref[0, 0]
    f = rij * cm
    ones = jnp.ones_like(cm)
    fvec_ref[...] = jnp.concatenate(
        [f, ones, jnp.zeros((f.shape[0], 12), jnp.float32)], axis=1)


def _edge_mlp(ga, gb, d, ef, w1n, w1ef, b1, w2, b2, cnw1, cnb1, cnw2, cnb2):
    e = ga.shape[0]
    te = 2000
    grid = (e // te,)
    full = lambda shp: pl.BlockSpec(shp, lambda i: (0, 0))
    de = ef.shape[1]
    return pl.pallas_call(
        _edge_kernel,
        grid=grid,
        in_specs=[
            pl.BlockSpec((te, H), lambda i: (i, 0)),
            pl.BlockSpec((te, H), lambda i: (i, 0)),
            pl.BlockSpec((te, 16), lambda i: (i, 0)),
            pl.BlockSpec((te, de), lambda i: (i, 0)),
            full((2, H)), full((de, H)), full((1, H)),
            full((H, H)), full((1, H)),
            full((H, H)), full((1, H)), full((H, 1)), full((1, 1)),
        ],
        out_specs=[
            pl.BlockSpec((te, H), lambda i: (i, 0)),
            pl.BlockSpec((te, 16), lambda i: (i, 0)),
        ],
        out_shape=[
            jax.ShapeDtypeStruct((e, H), jnp.float32),
            jax.ShapeDtypeStruct((e, 16), jnp.float32),
        ],
    )(ga, gb, d, ef, w1n, w1ef, b1, w2, b2, cnw1, cnb1, cnw2, cnb2)


def _fin_kernel(hpre_ref, accm0_ref, accm1_ref,
                nnw1b_ref, nnb1_ref, nnw2_ref, nnb2_ref, hnew_ref):
    accm = accm0_ref[0] + accm1_ref[0]
    t = _silu(hpre_ref[...]
              + jnp.dot(accm, nnw1b_ref[...],
                        preferred_element_type=jnp.float32)
              + nnb1_ref[...])
    hnew_ref[...] = jnp.dot(t, nnw2_ref[...],
                            preferred_element_type=jnp.float32) + nnb2_ref[...]


def _finalize(hpre, accm2, nnw1b, nnb1, nnw2, nnb2):
    n = hpre.shape[0]
    tn = 2000
    full = lambda shp: pl.BlockSpec(shp, lambda i: (0, 0))
    return pl.pallas_call(
        _fin_kernel,
        grid=(n // tn,),
        in_specs=[
            pl.BlockSpec((tn, H), lambda i: (i, 0)),
            pl.BlockSpec((1, tn, H), lambda i: (0, i, 0)),
            pl.BlockSpec((1, tn, H), lambda i: (1, i, 0)),
            full((H, H)), full((1, H)), full((H, H)), full((1, H)),
        ],
        out_specs=pl.BlockSpec((tn, H), lambda i: (i, 0)),
        out_shape=jax.ShapeDtypeStruct((n, H), jnp.float32),
    )(hpre, accm2, accm2, nnw1b, nnb1, nnw2, nnb2)


def kernel(x, h, edge_index, edge_fea, v,
           em_W1, em_b1, em_W2, em_b2,
           cn_W1, cn_b1, cn_W2, cn_b2,
           nn_W1, nn_b1, nn_W2, nn_b2,
           nx_W1, nx_b1, nx_W2, nx_b2,
           nv_W1, nv_b1, nv_W2, nv_b2):
    n = x.shape[0]
    row = edge_index[0]
    col = edge_index[1]

    w1n = em_W1[0:2]
    w1r = em_W1[2:2 + H]
    w1c = em_W1[2 + H:2 + 2 * H]
    w1ef = em_W1[2 + 2 * H:]
    nnw1t = nn_W1[:H]
    nnw1b = nn_W1[H:]

    ta, tb, hpre, gates = _precompute(
        h, w1r, w1c, nnw1t,
        nx_W1, nx_b1.reshape(1, H), nx_W2,
        nv_W1, nv_b1.reshape(1, H), nv_W2)
    gate_x = gates[:, 0:1] + nx_b2[0]
    gate_v = gates[:, 1:2] + nv_b2[0]

    xv = jnp.concatenate([x, v, jnp.zeros((n, DD - 22), jnp.float32)],
                         axis=1)
    ga, gb, d = _sc_gather(ta, tb, xv, row, col)

    msg, fvec = _edge_mlp(
        ga, gb, d, edge_fea, w1n, w1ef,
        em_b1.reshape(1, H), em_W2, em_b2.reshape(1, H),
        cn_W1, cn_b1.reshape(1, H), cn_W2, cn_b2.reshape(1, 1))

    zm = jnp.zeros((n, H), jnp.float32)
    accm2 = _sc_scatter(msg, row, zm)
    accf = jax.ops.segment_sum(fvec, row, num_segments=n)

    cnt = accf[:, 3:4]
    tot_f = jnp.clip(accf[:, 0:3] / jnp.clip(cnt, 1.0, None), -100.0, 100.0)

    v_new = gate_v * v + tot_f
    x_new = gate_x * x + tot_f

    h_new = _finalize(hpre, accm2, nnw1b,
                      nn_b1.reshape(1, H), nn_W2, nn_b2.reshape(1, H))
    return (x_new, v_new, h_new)


# speedup vs baseline: 5.0188x; 1.2875x over previous
import functools

import jax
import jax.numpy as jnp
from jax import lax
from jax.experimental import pallas as pl
from jax.experimental.pallas import tpu as pltpu
from jax.experimental.pallas import tpu_sc as plsc

H = 128
DD = 144
_NW = 32
_CH = 128


def _silu(u):
    return u * jax.nn.sigmoid(u)


def _pre_kernel(h_ref, w1r_ref, w1c_ref, nnw1t_ref,
                nxw1_ref, nxb1_ref, nxw2_ref,
                nvw1_ref, nvb1_ref, nvw2_ref,
                ta_ref, tb_ref, hpre_ref, g_ref):
    hh = h_ref[...]
    ta_ref[...] = jnp.dot(hh, w1r_ref[...], preferred_element_type=jnp.float32)
    tb_ref[...] = jnp.dot(hh, w1c_ref[...], preferred_element_type=jnp.float32)
    hpre_ref[...] = jnp.dot(hh, nnw1t_ref[...], preferred_element_type=jnp.float32)
    gx = _silu(jnp.dot(hh, nxw1_ref[...], preferred_element_type=jnp.float32)
               + nxb1_ref[...])
    gv = _silu(jnp.dot(hh, nvw1_ref[...], preferred_element_type=jnp.float32)
               + nvb1_ref[...])
    gxs = jnp.dot(gx, nxw2_ref[...], preferred_element_type=jnp.float32)
    gvs = jnp.dot(gv, nvw2_ref[...], preferred_element_type=jnp.float32)
    g_ref[...] = jnp.concatenate(
        [gxs, gvs, jnp.zeros_like(gxs), jnp.zeros_like(gvs)], axis=1)


def _precompute(h, w1r, w1c, nnw1t, nxw1, nxb1, nxw2, nvw1, nvb1, nvw2):
    n = h.shape[0]
    tn = 2000
    grid = (n // tn,)
    full = lambda shp: pl.BlockSpec(shp, lambda i: (0, 0))
    out = pl.pallas_call(
        _pre_kernel,
        grid=grid,
        in_specs=[
            pl.BlockSpec((tn, H), lambda i: (i, 0)),
            full((H, H)), full((H, H)), full((H, H)),
            full((H, H)), full((1, H)), full((H, 1)),
            full((H, H)), full((1, H)), full((H, 1)),
        ],
        out_specs=[
            pl.BlockSpec((tn, H), lambda i: (i, 0)),
            pl.BlockSpec((tn, H), lambda i: (i, 0)),
            pl.BlockSpec((tn, H), lambda i: (i, 0)),
            pl.BlockSpec((tn, 4), lambda i: (i, 0)),
        ],
        out_shape=[
            jax.ShapeDtypeStruct((n, H), jnp.float32),
            jax.ShapeDtypeStruct((n, H), jnp.float32),
            jax.ShapeDtypeStruct((n, H), jnp.float32),
            jax.ShapeDtypeStruct((n, 4), jnp.float32),
        ],
    )(h, w1r, w1c, nnw1t, nxw1, nxb1, nxw2, nvw1, nvb1, nvw2)
    return out


def _sc_gather(ta, tb, xv, row, col):
    n, dd = ta.shape
    e = row.shape[0]
    nchunk = e // _CH
    nloop = (nchunk + _NW - 1) // _NW
    mesh = plsc.VectorSubcoreMesh(core_axis_name="c", subcore_axis_name="s")

    @functools.partial(
        pl.kernel,
        out_type=[jax.ShapeDtypeStruct((e, dd), jnp.float32),
                  jax.ShapeDtypeStruct((e, dd), jnp.float32),
                  jax.ShapeDtypeStruct((e, 16), jnp.float32)],
        mesh=mesh,
        scratch_types=[
            pltpu.VMEM((_CH,), jnp.int32),
            pltpu.VMEM((_CH,), jnp.int32),
            pltpu.VMEM((_CH, dd), jnp.float32),
            pltpu.VMEM((_CH, dd), jnp.float32),
            pltpu.VMEM((_CH, dd), jnp.float32),
            pltpu.VMEM((_CH, dd), jnp.float32),
            pltpu.VMEM((_CH, 16), jnp.float32),
            pltpu.SemaphoreType.DMA,
            pltpu.SemaphoreType.DMA,
            pltpu.SemaphoreType.DMA,
            pltpu.SemaphoreType.DMA,
        ],
    )
    def k(ta_hbm, tb_hbm, xv_hbm, row_hbm, col_hbm, ga_hbm, gb_hbm, d_hbm,
          rowi, coli, abuf, bbuf, xr, xc, dbuf, sema, semb, semc, semd):
        wid = lax.axis_index("s") * 2 + lax.axis_index("c")

        @pl.loop(0, nloop)
        def _(i):
            c = wid + i * _NW

            @pl.when(c < nchunk)
            def _():
                base = c * _CH
                pltpu.sync_copy(row_hbm.at[pl.ds(base, _CH)], rowi)
                pltpu.sync_copy(col_hbm.at[pl.ds(base, _CH)], coli)
                ca = pltpu.async_copy(ta_hbm.at[rowi], abuf, sema)
                cb = pltpu.async_copy(tb_hbm.at[coli], bbuf, semb)
                cr = pltpu.async_copy(xv_hbm.at[rowi], xr, semc)
                cc = pltpu.async_copy(xv_hbm.at[coli], xc, semd)
                cr.wait()
                cc.wait()
                for r in range(_CH):
                    dbuf[r, :] = xr[r, pl.ds(0, 16)] - xc[r, pl.ds(0, 16)]
                ca.wait()
                cb.wait()
                pltpu.sync_copy(abuf, ga_hbm.at[pl.ds(base, _CH)])
                pltpu.sync_copy(bbuf, gb_hbm.at[pl.ds(base, _CH)])
                pltpu.sync_copy(dbuf, d_hbm.at[pl.ds(base, _CH)])

    return k(ta, tb, xv, row, col)


def _sc_scatter(msg, fvec, row, zm):
    e, hh = msg.shape
    n = zm.shape[0]
    nchunk = e // _CH
    nsub = 16
    nloop = (nchunk + nsub - 1) // nsub
    mesh = plsc.VectorSubcoreMesh(core_axis_name="c", subcore_axis_name="s")

    @functools.partial(
        pl.kernel,
        out_type=jax.ShapeDtypeStruct((2, n, hh), jnp.float32),
        mesh=mesh,
        scratch_types=[
            pltpu.VMEM((_CH,), jnp.int32),
            pltpu.VMEM((_CH, hh), jnp.float32),
            pltpu.VMEM((_CH, 16), jnp.float32),
            pltpu.VMEM_SHARED((n, hh), jnp.float32),
        ],
    )
    def k(msg_hbm, fvec_hbm, row_hbm, zm_hbm, out_hbm, rowi, mbuf, fbuf, acc):
        c = lax.axis_index("c")
        s = lax.axis_index("s")
        sz_a, sz_b = 624, n - 15 * 624

        @pl.when(s < 15)
        def _():
            r0 = pl.multiple_of(s * sz_a, 8)
            pltpu.sync_copy(zm_hbm.at[pl.ds(r0, sz_a)], acc.at[pl.ds(r0, sz_a)])

        @pl.when(s == 15)
        def _():
            pltpu.sync_copy(zm_hbm.at[pl.ds(15 * sz_a, sz_b)],
                            acc.at[pl.ds(15 * sz_a, sz_b)])

        @pl.when(c == 1)
        def _():
            pltpu.sync_copy(zm_hbm.at[pl.ds(0, _CH)], mbuf)

        plsc.subcore_barrier()

        @pl.loop(0, nloop)
        def _(i):
            j = s + i * nsub

            @pl.when(jnp.logical_and(j < nchunk, c == 0))
            def _():
                base = j * _CH
                pltpu.sync_copy(row_hbm.at[pl.ds(base, _CH)], rowi)
                pltpu.sync_copy(msg_hbm.at[pl.ds(base, _CH)], mbuf)
                pltpu.sync_copy(mbuf, acc.at[rowi], add=True)

            @pl.when(jnp.logical_and(j < nchunk, c == 1))
            def _():
                base = j * _CH
                pltpu.sync_copy(row_hbm.at[pl.ds(base, _CH)], rowi)
                pltpu.sync_copy(fvec_hbm.at[pl.ds(base, _CH)], fbuf)
                for r in range(_CH):
                    mbuf[r, pl.ds(0, 16)] = fbuf[r, :]
                pltpu.sync_copy(mbuf, acc.at[rowi], add=True)

        plsc.subcore_barrier()

        @pl.when(s < 15)
        def _():
            r0 = pl.multiple_of(s * sz_a, 8)
            pltpu.sync_copy(acc.at[pl.ds(r0, sz_a)],
                            out_hbm.at[c, pl.ds(r0, sz_a)])

        @pl.when(s == 15)
        def _():
            pltpu.sync_copy(acc.at[pl.ds(15 * sz_a, sz_b)],
                            out_hbm.at[c, pl.ds(15 * sz_a, sz_b)])

    return k(msg, fvec, row, zm)


def _edge_kernel(ga_ref, gb_ref, d_ref, ef_ref,
                 w1n_ref, w1ef_ref, b1_ref, w2_ref, b2_ref,
                 cnw1_ref, cnb1_ref, cnw2_ref, cnb2_ref,
                 msg_ref, fvec_ref):
    ga = ga_ref[...]
    gb = gb_ref[...]
    d = d_ref[...]
    rij = d[:, 0:3]
    vij = d[:, 3:6]
    nr = jnp.sqrt(jnp.sum(rij * rij, axis=1, keepdims=True))
    nv = jnp.sqrt(jnp.sum(vij * vij, axis=1, keepdims=True))
    w1n = w1n_ref[...]
    pre1 = (ga + gb + b1_ref[...]
            + nr * w1n[0:1, :] + nv * w1n[1:2, :]
            + jnp.dot(ef_ref[...], w1ef_ref[...],
                      preferred_element_type=jnp.float32))
    m1 = _silu(pre1)
    msg = _silu(jnp.dot(m1, w2_ref[...], preferred_element_type=jnp.float32)
                + b2_ref[...])
    msg_ref[...] = msg
    ch = _silu(jnp.dot(msg, cnw1_ref[...], preferred_element_type=jnp.float32)
               + cnb1_ref[...])
    cm = jnp.dot(ch, cnw2_ref[...], preferred_element_type=jnp.float32) \
        + cnb2_ref[0, 0]
    f = rij * cm
    ones = jnp.ones_like(cm)
    fvec_ref[...] = jnp.concatenate(
        [f, ones, jnp.zeros((f.shape[0], 12), jnp.float32)], axis=1)


def _edge_mlp(ga, gb, d, ef, w1n, w1ef, b1, w2, b2, cnw1, cnb1, cnw2, cnb2):
    e = ga.shape[0]
    te = 2000
    grid = (e // te,)
    full = lambda shp: pl.BlockSpec(shp, lambda i: (0, 0))
    de = ef.shape[1]
    return pl.pallas_call(
        _edge_kernel,
        grid=grid,
        in_specs=[
            pl.BlockSpec((te, H), lambda i: (i, 0)),
            pl.BlockSpec((te, H), lambda i: (i, 0)),
            pl.BlockSpec((te, 16), lambda i: (i, 0)),
            pl.BlockSpec((te, de), lambda i: (i, 0)),
            full((2, H)), full((de, H)), full((1, H)),
            full((H, H)), full((1, H)),
            full((H, H)), full((1, H)), full((H, 1)), full((1, 1)),
        ],
        out_specs=[
            pl.BlockSpec((te, H), lambda i: (i, 0)),
            pl.BlockSpec((te, 16), lambda i: (i, 0)),
        ],
        out_shape=[
            jax.ShapeDtypeStruct((e, H), jnp.float32),
            jax.ShapeDtypeStruct((e, 16), jnp.float32),
        ],
    )(ga, gb, d, ef, w1n, w1ef, b1, w2, b2, cnw1, cnb1, cnw2, cnb2)


def _fin_kernel(hpre_ref, accm0_ref,
                nnw1b_ref, nnb1_ref, nnw2_ref, nnb2_ref, hnew_ref):
    accm = accm0_ref[0]
    t = _silu(hpre_ref[...]
              + jnp.dot(accm, nnw1b_ref[...],
                        preferred_element_type=jnp.float32)
              + nnb1_ref[...])
    hnew_ref[...] = jnp.dot(t, nnw2_ref[...],
                            preferred_element_type=jnp.float32) + nnb2_ref[...]


def _finalize(hpre, accm2, nnw1b, nnb1, nnw2, nnb2):
    n = hpre.shape[0]
    tn = 2000
    full = lambda shp: pl.BlockSpec(shp, lambda i: (0, 0))
    return pl.pallas_call(
        _fin_kernel,
        grid=(n // tn,),
        in_specs=[
            pl.BlockSpec((tn, H), lambda i: (i, 0)),
            pl.BlockSpec((1, tn, H), lambda i: (0, i, 0)),
            full((H, H)), full((1, H)), full((H, H)), full((1, H)),
        ],
        out_specs=pl.BlockSpec((tn, H), lambda i: (i, 0)),
        out_shape=jax.ShapeDtypeStruct((n, H), jnp.float32),
    )(hpre, accm2, nnw1b, nnb1, nnw2, nnb2)


def kernel(x, h, edge_index, edge_fea, v,
           em_W1, em_b1, em_W2, em_b2,
           cn_W1, cn_b1, cn_W2, cn_b2,
           nn_W1, nn_b1, nn_W2, nn_b2,
           nx_W1, nx_b1, nx_W2, nx_b2,
           nv_W1, nv_b1, nv_W2, nv_b2):
    n = x.shape[0]
    row = edge_index[0]
    col = edge_index[1]

    w1n = em_W1[0:2]
    w1r = em_W1[2:2 + H]
    w1c = em_W1[2 + H:2 + 2 * H]
    w1ef = em_W1[2 + 2 * H:]
    nnw1t = nn_W1[:H]
    nnw1b = nn_W1[H:]

    ta, tb, hpre, gates = _precompute(
        h, w1r, w1c, nnw1t,
        nx_W1, nx_b1.reshape(1, H), nx_W2,
        nv_W1, nv_b1.reshape(1, H), nv_W2)
    gate_x = gates[:, 0:1] + nx_b2[0]
    gate_v = gates[:, 1:2] + nv_b2[0]

    xv = jnp.concatenate([x, v, jnp.zeros((n, DD - 22), jnp.float32)],
                         axis=1)
    ga, gb, d = _sc_gather(ta, tb, xv, row, col)

    msg, fvec = _edge_mlp(
        ga, gb, d, edge_fea, w1n, w1ef,
        em_b1.reshape(1, H), em_W2, em_b2.reshape(1, H),
        cn_W1, cn_b1.reshape(1, H), cn_W2, cn_b2.reshape(1, 1))

    zm = jnp.zeros((n, H), jnp.float32)
    acc2 = _sc_scatter(msg, fvec, row, zm)
    accf = acc2[1, :, :16]

    cnt = accf[:, 3:4]
    tot_f = jnp.clip(accf[:, 0:3] / jnp.clip(cnt, 1.0, None), -100.0, 100.0)

    v_new = gate_v * v + tot_f
    x_new = gate_x * x + tot_f

    h_new = _finalize(hpre, acc2, nnw1b,
                      nn_b1.reshape(1, H), nn_W2, nn_b2.reshape(1, H))
    return (x_new, v_new, h_new)


# speedup vs baseline: 5.2751x; 1.0511x over previous
import functools

import jax
import jax.numpy as jnp
from jax import lax
from jax.experimental import pallas as pl
from jax.experimental.pallas import tpu as pltpu
from jax.experimental.pallas import tpu_sc as plsc

H = 128
DD = 144
_NW = 32
_CH = 128


def _silu(u):
    return u * jax.nn.sigmoid(u)


def _pre_kernel(h_ref, w1r_ref, w1c_ref, nnw1t_ref,
                nxw1_ref, nxb1_ref, nxw2_ref,
                nvw1_ref, nvb1_ref, nvw2_ref,
                ta_ref, tb_ref, hpre_ref, g_ref):
    hh = h_ref[...]
    ta_ref[...] = jnp.dot(hh, w1r_ref[...], preferred_element_type=jnp.float32)
    tb_ref[...] = jnp.dot(hh, w1c_ref[...], preferred_element_type=jnp.float32)
    hpre_ref[...] = jnp.dot(hh, nnw1t_ref[...], preferred_element_type=jnp.float32)
    gx = _silu(jnp.dot(hh, nxw1_ref[...], preferred_element_type=jnp.float32)
               + nxb1_ref[...])
    gv = _silu(jnp.dot(hh, nvw1_ref[...], preferred_element_type=jnp.float32)
               + nvb1_ref[...])
    gxs = jnp.dot(gx, nxw2_ref[...], preferred_element_type=jnp.float32)
    gvs = jnp.dot(gv, nvw2_ref[...], preferred_element_type=jnp.float32)
    g_ref[...] = jnp.concatenate(
        [gxs, gvs, jnp.zeros_like(gxs), jnp.zeros_like(gvs)], axis=1)


def _precompute(h, w1r, w1c, nnw1t, nxw1, nxb1, nxw2, nvw1, nvb1, nvw2):
    n = h.shape[0]
    tn = 2000
    grid = (n // tn,)
    full = lambda shp: pl.BlockSpec(shp, lambda i: (0, 0))
    out = pl.pallas_call(
        _pre_kernel,
        grid=grid,
        in_specs=[
            pl.BlockSpec((tn, H), lambda i: (i, 0)),
            full((H, H)), full((H, H)), full((H, H)),
            full((H, H)), full((1, H)), full((H, 1)),
            full((H, H)), full((1, H)), full((H, 1)),
        ],
        out_specs=[
            pl.BlockSpec((tn, H), lambda i: (i, 0)),
            pl.BlockSpec((tn, H), lambda i: (i, 0)),
            pl.BlockSpec((tn, H), lambda i: (i, 0)),
            pl.BlockSpec((tn, 4), lambda i: (i, 0)),
        ],
        out_shape=[
            jax.ShapeDtypeStruct((n, H), jnp.float32),
            jax.ShapeDtypeStruct((n, H), jnp.float32),
            jax.ShapeDtypeStruct((n, H), jnp.float32),
            jax.ShapeDtypeStruct((n, 4), jnp.float32),
        ],
    )(h, w1r, w1c, nnw1t, nxw1, nxb1, nxw2, nvw1, nvb1, nvw2)
    return out


def _sc_gather(ta, tb, xv, row, col):
    n, dd = ta.shape
    e = row.shape[0]
    nchunk = e // _CH
    nloop = (nchunk + _NW - 1) // _NW
    mesh = plsc.VectorSubcoreMesh(core_axis_name="c", subcore_axis_name="s")

    @functools.partial(
        pl.kernel,
        out_type=[jax.ShapeDtypeStruct((e, dd), jnp.float32),
                  jax.ShapeDtypeStruct((e, dd), jnp.float32),
                  jax.ShapeDtypeStruct((e, 16), jnp.float32)],
        mesh=mesh,
        scratch_types=[
            pltpu.VMEM((_CH,), jnp.int32),
            pltpu.VMEM((_CH,), jnp.int32),
            pltpu.VMEM((_CH, dd), jnp.float32),
            pltpu.VMEM((_CH, dd), jnp.float32),
            pltpu.VMEM((_CH, dd), jnp.float32),
            pltpu.VMEM((_CH, dd), jnp.float32),
            pltpu.VMEM((_CH, dd), jnp.float32),
            pltpu.VMEM((_CH, dd), jnp.float32),
            pltpu.VMEM((_CH, 16), jnp.float32),
            pltpu.SemaphoreType.DMA,
            pltpu.SemaphoreType.DMA,
            pltpu.SemaphoreType.DMA,
            pltpu.SemaphoreType.DMA,
        ],
    )
    def k(ta_hbm, tb_hbm, xv_hbm, row_hbm, col_hbm, ga_hbm, gb_hbm, d_hbm,
          rowi, coli, abuf0, abuf1, bbuf0, bbuf1, xr, xc, dbuf,
          sema, semb, semc, semd):
        wid = lax.axis_index("s") * 2 + lax.axis_index("c")
        ab = (abuf0, abuf1)
        bb = (bbuf0, bbuf1)

        def load_idx(c):
            base = c * _CH
            pltpu.sync_copy(row_hbm.at[pl.ds(base, _CH)], rowi)
            pltpu.sync_copy(col_hbm.at[pl.ds(base, _CH)], coli)

        def issue(p):
            pltpu.async_copy(ta_hbm.at[rowi], ab[p], sema)
            pltpu.async_copy(tb_hbm.at[coli], bb[p], semb)
            pltpu.async_copy(xv_hbm.at[rowi], xr, semc)
            pltpu.async_copy(xv_hbm.at[coli], xc, semd)

        def wait(p):
            pltpu.make_async_copy(ta_hbm.at[rowi], ab[p], sema).wait()
            pltpu.make_async_copy(tb_hbm.at[coli], bb[p], semb).wait()
            pltpu.make_async_copy(xv_hbm.at[rowi], xr, semc).wait()
            pltpu.make_async_copy(xv_hbm.at[coli], xc, semd).wait()

        def step(c, p):
            @pl.when(c < nchunk)
            def _():
                wait(p)
                for r in range(_CH):
                    dbuf[r, :] = xr[r, pl.ds(0, 16)] - xc[r, pl.ds(0, 16)]

                @pl.when(c + _NW < nchunk)
                def _():
                    load_idx(c + _NW)
                    issue(1 - p)

                base = c * _CH
                pltpu.sync_copy(ab[p], ga_hbm.at[pl.ds(base, _CH)])
                pltpu.sync_copy(bb[p], gb_hbm.at[pl.ds(base, _CH)])
                pltpu.sync_copy(dbuf, d_hbm.at[pl.ds(base, _CH)])

        load_idx(wid)
        issue(0)

        @pl.loop(0, (nloop + 1) // 2)
        def _(i):
            step(wid + (2 * i) * _NW, 0)
            step(wid + (2 * i + 1) * _NW, 1)

    return k(ta, tb, xv, row, col)


def _sc_scatter(msg, fvec, row, zm):
    e, hh = msg.shape
    n = zm.shape[0]
    nchunk = e // _CH
    nsub = 16
    nloop = (nchunk + nsub - 1) // nsub
    mesh = plsc.VectorSubcoreMesh(core_axis_name="c", subcore_axis_name="s")

    @functools.partial(
        pl.kernel,
        out_type=jax.ShapeDtypeStruct((2, n, hh), jnp.float32),
        mesh=mesh,
        scratch_types=[
            pltpu.VMEM((_CH,), jnp.int32),
            pltpu.VMEM((_CH, hh), jnp.float32),
            pltpu.VMEM((_CH, 16), jnp.float32),
            pltpu.VMEM_SHARED((n, hh), jnp.float32),
        ],
    )
    def k(msg_hbm, fvec_hbm, row_hbm, zm_hbm, out_hbm, rowi, mbuf, fbuf, acc):
        c = lax.axis_index("c")
        s = lax.axis_index("s")
        sz_a, sz_b = 624, n - 15 * 624

        @pl.when(s < 15)
        def _():
            r0 = pl.multiple_of(s * sz_a, 8)
            pltpu.sync_copy(zm_hbm.at[pl.ds(r0, sz_a)], acc.at[pl.ds(r0, sz_a)])

        @pl.when(s == 15)
        def _():
            pltpu.sync_copy(zm_hbm.at[pl.ds(15 * sz_a, sz_b)],
                            acc.at[pl.ds(15 * sz_a, sz_b)])

        @pl.when(c == 1)
        def _():
            pltpu.sync_copy(zm_hbm.at[pl.ds(0, _CH)], mbuf)

        plsc.subcore_barrier()

        @pl.loop(0, nloop)
        def _(i):
            j = s + i * nsub

            @pl.when(jnp.logical_and(j < nchunk, c == 0))
            def _():
                base = j * _CH
                pltpu.sync_copy(row_hbm.at[pl.ds(base, _CH)], rowi)
                pltpu.sync_copy(msg_hbm.at[pl.ds(base, _CH)], mbuf)
                pltpu.sync_copy(mbuf, acc.at[rowi], add=True)

            @pl.when(jnp.logical_and(j < nchunk, c == 1))
            def _():
                base = j * _CH
                pltpu.sync_copy(row_hbm.at[pl.ds(base, _CH)], rowi)
                pltpu.sync_copy(fvec_hbm.at[pl.ds(base, _CH)], fbuf)
                for r in range(_CH):
                    mbuf[r, pl.ds(0, 16)] = fbuf[r, :]
                pltpu.sync_copy(mbuf, acc.at[rowi], add=True)

        plsc.subcore_barrier()

        @pl.when(s < 15)
        def _():
            r0 = pl.multiple_of(s * sz_a, 8)
            pltpu.sync_copy(acc.at[pl.ds(r0, sz_a)],
                            out_hbm.at[c, pl.ds(r0, sz_a)])

        @pl.when(s == 15)
        def _():
            pltpu.sync_copy(acc.at[pl.ds(15 * sz_a, sz_b)],
                            out_hbm.at[c, pl.ds(15 * sz_a, sz_b)])

    return k(msg, fvec, row, zm)


def _edge_kernel(ga_ref, gb_ref, d_ref, ef_ref,
                 w1n_ref, w1ef_ref, b1_ref, w2_ref, b2_ref,
                 cnw1_ref, cnb1_ref, cnw2_ref, cnb2_ref,
                 msg_ref, fvec_ref):
    ga = ga_ref[...]
    gb = gb_ref[...]
    d = d_ref[...]
    rij = d[:, 0:3]
    vij = d[:, 3:6]
    nr = jnp.sqrt(jnp.sum(rij * rij, axis=1, keepdims=True))
    nv = jnp.sqrt(jnp.sum(vij * vij, axis=1, keepdims=True))
    w1n = w1n_ref[...]
    pre1 = (ga + gb + b1_ref[...]
            + nr * w1n[0:1, :] + nv * w1n[1:2, :]
            + jnp.dot(ef_ref[...], w1ef_ref[...],
                      preferred_element_type=jnp.float32))
    m1 = _silu(pre1)
    msg = _silu(jnp.dot(m1, w2_ref[...], preferred_element_type=jnp.float32)
                + b2_ref[...])
    msg_ref[...] = msg
    ch = _silu(jnp.dot(msg, cnw1_ref[...], preferred_element_type=jnp.float32)
               + cnb1_ref[...])
    cm = jnp.dot(ch, cnw2_ref[...], preferred_element_type=jnp.float32) \
        + cnb2_ref[0, 0]
    f = rij * cm
    ones = jnp.ones_like(cm)
    fvec_ref[...] = jnp.concatenate(
        [f, ones, jnp.zeros((f.shape[0], 12), jnp.float32)], axis=1)


def _edge_mlp(ga, gb, d, ef, w1n, w1ef, b1, w2, b2, cnw1, cnb1, cnw2, cnb2):
    e = ga.shape[0]
    te = 2000
    grid = (e // te,)
    full = lambda shp: pl.BlockSpec(shp, lambda i: (0, 0))
    de = ef.shape[1]
    return pl.pallas_call(
        _edge_kernel,
        grid=grid,
        in_specs=[
            pl.BlockSpec((te, H), lambda i: (i, 0)),
            pl.BlockSpec((te, H), lambda i: (i, 0)),
            pl.BlockSpec((te, 16), lambda i: (i, 0)),
            pl.BlockSpec((te, de), lambda i: (i, 0)),
            full((2, H)), full((de, H)), full((1, H)),
            full((H, H)), full((1, H)),
            full((H, H)), full((1, H)), full((H, 1)), full((1, 1)),
        ],
        out_specs=[
            pl.BlockSpec((te, H), lambda i: (i, 0)),
            pl.BlockSpec((te, 16), lambda i: (i, 0)),
        ],
        out_shape=[
            jax.ShapeDtypeStruct((e, H), jnp.float32),
            jax.ShapeDtypeStruct((e, 16), jnp.float32),
        ],
    )(ga, gb, d, ef, w1n, w1ef, b1, w2, b2, cnw1, cnb1, cnw2, cnb2)


def _fin_kernel(hpre_ref, accm0_ref,
                nnw1b_ref, nnb1_ref, nnw2_ref, nnb2_ref, hnew_ref):
    accm = accm0_ref[0]
    t = _silu(hpre_ref[...]
              + jnp.dot(accm, nnw1b_ref[...],
                        preferred_element_type=jnp.float32)
              + nnb1_ref[...])
    hnew_ref[...] = jnp.dot(t, nnw2_ref[...],
                            preferred_element_type=jnp.float32) + nnb2_ref[...]


def _finalize(hpre, accm2, nnw1b, nnb1, nnw2, nnb2):
    n = hpre.shape[0]
    tn = 2000
    full = lambda shp: pl.BlockSpec(shp, lambda i: (0, 0))
    return pl.pallas_call(
        _fin_kernel,
        grid=(n // tn,),
        in_specs=[
            pl.BlockSpec((tn, H), lambda i: (i, 0)),
            pl.BlockSpec((1, tn, H), lambda i: (0, i, 0)),
            full((H, H)), full((1, H)), full((H, H)), full((1, H)),
        ],
        out_specs=pl.BlockSpec((tn, H), lambda i: (i, 0)),
        out_shape=jax.ShapeDtypeStruct((n, H), jnp.float32),
    )(hpre, accm2, nnw1b, nnb1, nnw2, nnb2)


def kernel(x, h, edge_index, edge_fea, v,
           em_W1, em_b1, em_W2, em_b2,
           cn_W1, cn_b1, cn_W2, cn_b2,
           nn_W1, nn_b1, nn_W2, nn_b2,
           nx_W1, nx_b1, nx_W2, nx_b2,
           nv_W1, nv_b1, nv_W2, nv_b2):
    n = x.shape[0]
    row = edge_index[0]
    col = edge_index[1]

    w1n = em_W1[0:2]
    w1r = em_W1[2:2 + H]
    w1c = em_W1[2 + H:2 + 2 * H]
    w1ef = em_W1[2 + 2 * H:]
    nnw1t = nn_W1[:H]
    nnw1b = nn_W1[H:]

    ta, tb, hpre, gates = _precompute(
        h, w1r, w1c, nnw1t,
        nx_W1, nx_b1.reshape(1, H), nx_W2,
        nv_W1, nv_b1.reshape(1, H), nv_W2)
    gate_x = gates[:, 0:1] + nx_b2[0]
    gate_v = gates[:, 1:2] + nv_b2[0]

    xv = jnp.concatenate([x, v, jnp.zeros((n, DD - 22), jnp.float32)],
                         axis=1)
    ga, gb, d = _sc_gather(ta, tb, xv, row, col)

    msg, fvec = _edge_mlp(
        ga, gb, d, edge_fea, w1n, w1ef,
        em_b1.reshape(1, H), em_W2, em_b2.reshape(1, H),
        cn_W1, cn_b1.reshape(1, H), cn_W2, cn_b2.reshape(1, 1))

    zm = jnp.zeros((n, H), jnp.float32)
    acc2 = _sc_scatter(msg, fvec, row, zm)
    accf = acc2[1, :, :16]

    cnt = accf[:, 3:4]
    tot_f = jnp.clip(accf[:, 0:3] / jnp.clip(cnt, 1.0, None), -100.0, 100.0)

    v_new = gate_v * v + tot_f
    x_new = gate_x * x + tot_f

    h_new = _finalize(hpre, acc2, nnw1b,
                      nn_b1.reshape(1, H), nn_W2, nn_b2.reshape(1, H))
    return (x_new, v_new, h_new)


# speedup vs baseline: 5.6105x; 1.0636x over previous
import functools

import jax
import jax.numpy as jnp
from jax import lax
from jax.experimental import pallas as pl
from jax.experimental.pallas import tpu as pltpu
from jax.experimental.pallas import tpu_sc as plsc

H = 128
DD = 144
_NW = 32
_CH = 128


def _silu(u):
    return u * jax.nn.sigmoid(u)


def _pre_kernel(h_ref, w1r_ref, w1c_ref, nnw1t_ref,
                nxw1_ref, nxb1_ref, nxw2_ref,
                nvw1_ref, nvb1_ref, nvw2_ref,
                ta_ref, tb_ref, hpre_ref, g_ref):
    hh = h_ref[...]
    ta_ref[...] = jnp.dot(hh, w1r_ref[...], preferred_element_type=jnp.float32)
    tb_ref[...] = jnp.dot(hh, w1c_ref[...], preferred_element_type=jnp.float32)
    hpre_ref[...] = jnp.dot(hh, nnw1t_ref[...], preferred_element_type=jnp.float32)
    gx = _silu(jnp.dot(hh, nxw1_ref[...], preferred_element_type=jnp.float32)
               + nxb1_ref[...])
    gv = _silu(jnp.dot(hh, nvw1_ref[...], preferred_element_type=jnp.float32)
               + nvb1_ref[...])
    gxs = jnp.dot(gx, nxw2_ref[...], preferred_element_type=jnp.float32)
    gvs = jnp.dot(gv, nvw2_ref[...], preferred_element_type=jnp.float32)
    g_ref[...] = jnp.concatenate(
        [gxs, gvs, jnp.zeros_like(gxs), jnp.zeros_like(gvs)], axis=1)


def _precompute(h, w1r, w1c, nnw1t, nxw1, nxb1, nxw2, nvw1, nvb1, nvw2):
    n = h.shape[0]
    tn = 2000
    grid = (n // tn,)
    full = lambda shp: pl.BlockSpec(shp, lambda i: (0, 0))
    out = pl.pallas_call(
        _pre_kernel,
        grid=grid,
        in_specs=[
            pl.BlockSpec((tn, H), lambda i: (i, 0)),
            full((H, H)), full((H, H)), full((H, H)),
            full((H, H)), full((1, H)), full((H, 1)),
            full((H, H)), full((1, H)), full((H, 1)),
        ],
        out_specs=[
            pl.BlockSpec((tn, H), lambda i: (i, 0)),
            pl.BlockSpec((tn, H), lambda i: (i, 0)),
            pl.BlockSpec((tn, H), lambda i: (i, 0)),
            pl.BlockSpec((tn, 4), lambda i: (i, 0)),
        ],
        out_shape=[
            jax.ShapeDtypeStruct((n, H), jnp.float32),
            jax.ShapeDtypeStruct((n, H), jnp.float32),
            jax.ShapeDtypeStruct((n, H), jnp.float32),
            jax.ShapeDtypeStruct((n, 4), jnp.float32),
        ],
    )(h, w1r, w1c, nnw1t, nxw1, nxb1, nxw2, nvw1, nvb1, nvw2)
    return out


def _sc_gather(ta, tb, xv, row, col):
    n, dd = ta.shape
    e = row.shape[0]
    nchunk = e // _CH
    nloop = (nchunk + _NW - 1) // _NW
    mesh = plsc.VectorSubcoreMesh(core_axis_name="c", subcore_axis_name="s")

    @functools.partial(
        pl.kernel,
        out_type=[jax.ShapeDtypeStruct((e, dd), jnp.float32),
                  jax.ShapeDtypeStruct((e, dd), jnp.float32),
                  jax.ShapeDtypeStruct((e, 16), jnp.float32)],
        mesh=mesh,
        scratch_types=[
            pltpu.VMEM((_CH,), jnp.int32),
            pltpu.VMEM((_CH,), jnp.int32),
            pltpu.VMEM((_CH, dd), jnp.float32),
            pltpu.VMEM((_CH, dd), jnp.float32),
            pltpu.VMEM((_CH, dd), jnp.float32),
            pltpu.VMEM((_CH, dd), jnp.float32),
            pltpu.VMEM((_CH, dd), jnp.float32),
            pltpu.VMEM((_CH, dd), jnp.float32),
            pltpu.VMEM((_CH, 16), jnp.float32),
            pltpu.SemaphoreType.DMA,
            pltpu.SemaphoreType.DMA,
            pltpu.SemaphoreType.DMA,
            pltpu.SemaphoreType.DMA,
        ],
    )
    def k(ta_hbm, tb_hbm, xv_hbm, row_hbm, col_hbm, ga_hbm, gb_hbm, d_hbm,
          rowi, coli, abuf0, abuf1, bbuf0, bbuf1, xr, xc, dbuf,
          sema, semb, semc, semd):
        wid = lax.axis_index("s") * 2 + lax.axis_index("c")
        ab = (abuf0, abuf1)
        bb = (bbuf0, bbuf1)

        def load_idx(c):
            base = c * _CH
            pltpu.sync_copy(row_hbm.at[pl.ds(base, _CH)], rowi)
            pltpu.sync_copy(col_hbm.at[pl.ds(base, _CH)], coli)

        def issue(p):
            pltpu.async_copy(ta_hbm.at[rowi], ab[p], sema)
            pltpu.async_copy(tb_hbm.at[coli], bb[p], semb)
            pltpu.async_copy(xv_hbm.at[rowi], xr, semc)
            pltpu.async_copy(xv_hbm.at[coli], xc, semd)

        def wait(p):
            pltpu.make_async_copy(ta_hbm.at[rowi], ab[p], sema).wait()
            pltpu.make_async_copy(tb_hbm.at[coli], bb[p], semb).wait()
            pltpu.make_async_copy(xv_hbm.at[rowi], xr, semc).wait()
            pltpu.make_async_copy(xv_hbm.at[coli], xc, semd).wait()

        def step(c, p):
            @pl.when(c < nchunk)
            def _():
                wait(p)
                for r in range(_CH):
                    dbuf[r, :] = xr[r, pl.ds(0, 16)] - xc[r, pl.ds(0, 16)]

                @pl.when(c + _NW < nchunk)
                def _():
                    load_idx(c + _NW)
                    issue(1 - p)

                base = c * _CH
                pltpu.sync_copy(ab[p], ga_hbm.at[pl.ds(base, _CH)])
                pltpu.sync_copy(bb[p], gb_hbm.at[pl.ds(base, _CH)])
                pltpu.sync_copy(dbuf, d_hbm.at[pl.ds(base, _CH)])

        load_idx(wid)
        issue(0)

        @pl.loop(0, (nloop + 1) // 2)
        def _(i):
            step(wid + (2 * i) * _NW, 0)
            step(wid + (2 * i + 1) * _NW, 1)

    return k(ta, tb, xv, row, col)


def _sc_scatter(msg, fvec, row, zm):
    e, hh = msg.shape
    n = zm.shape[0]
    chs = 64
    nchunk = e // chs
    nsub = 16
    nloop = (nchunk + nsub - 1) // nsub
    mesh = plsc.VectorSubcoreMesh(core_axis_name="c", subcore_axis_name="s")

    @functools.partial(
        pl.kernel,
        out_type=jax.ShapeDtypeStruct((2, n, hh), jnp.float32),
        mesh=mesh,
        scratch_types=[
            pltpu.VMEM((chs,), jnp.int32),
            pltpu.VMEM((chs,), jnp.int32),
            pltpu.VMEM((chs, hh), jnp.float32),
            pltpu.VMEM((chs, hh), jnp.float32),
            pltpu.VMEM((chs, 16), jnp.float32),
            pltpu.VMEM((chs, 16), jnp.float32),
            pltpu.VMEM_SHARED((n, hh), jnp.float32),
            pltpu.SemaphoreType.DMA,
            pltpu.SemaphoreType.DMA,
            pltpu.SemaphoreType.DMA,
        ],
    )
    def k(msg_hbm, fvec_hbm, row_hbm, zm_hbm, out_hbm,
          rowi0, rowi1, mbuf0, mbuf1, fbuf0, fbuf1, acc,
          semr, semm, sems):
        c = lax.axis_index("c")
        s = lax.axis_index("s")
        ri = (rowi0, rowi1)
        mb = (mbuf0, mbuf1)
        fb = (fbuf0, fbuf1)
        sz_a, sz_b = 624, n - 15 * 624

        @pl.when(s < 15)
        def _():
            r0 = pl.multiple_of(s * sz_a, 8)
            pltpu.sync_copy(zm_hbm.at[pl.ds(r0, sz_a)], acc.at[pl.ds(r0, sz_a)])

        @pl.when(s == 15)
        def _():
            pltpu.sync_copy(zm_hbm.at[pl.ds(15 * sz_a, sz_b)],
                            acc.at[pl.ds(15 * sz_a, sz_b)])

        @pl.when(c == 1)
        def _():
            pltpu.sync_copy(zm_hbm.at[pl.ds(0, chs)], mbuf0)
            pltpu.sync_copy(zm_hbm.at[pl.ds(0, chs)], mbuf1)

        plsc.subcore_barrier()

        def issue_loads(j, p):
            base = j * chs
            pltpu.async_copy(row_hbm.at[pl.ds(base, chs)], ri[p], semr)

            @pl.when(c == 0)
            def _():
                pltpu.async_copy(msg_hbm.at[pl.ds(base, chs)], mb[p], semm)

            @pl.when(c == 1)
            def _():
                pltpu.async_copy(fvec_hbm.at[pl.ds(base, chs)], fb[p], semm)

        def wait_loads(p):
            pltpu.make_async_copy(row_hbm.at[pl.ds(0, chs)], ri[p],
                                  semr).wait()

            @pl.when(c == 0)
            def _():
                pltpu.make_async_copy(msg_hbm.at[pl.ds(0, chs)], mb[p],
                                      semm).wait()

            @pl.when(c == 1)
            def _():
                pltpu.make_async_copy(fvec_hbm.at[pl.ds(0, chs)], fb[p],
                                      semm).wait()

        def wait_scatter(p):
            pltpu.make_async_copy(mb[p], acc.at[ri[p]], sems).wait()

        def step(j, p):
            @pl.when(j < nchunk)
            def _():
                wait_loads(p)

                @pl.when(j > s)
                def _():
                    wait_scatter(1 - p)

                @pl.when(j + nsub < nchunk)
                def _():
                    issue_loads(j + nsub, 1 - p)

                @pl.when(c == 1)
                def _():
                    for r in range(chs):
                        mb[p][r, pl.ds(0, 16)] = fb[p][r, :]

                pltpu.async_copy(mb[p], acc.at[ri[p]], sems, add=True)

        issue_loads(s, 0)

        @pl.loop(0, (nloop + 1) // 2)
        def _(i):
            step(s + (2 * i) * nsub, 0)
            step(s + (2 * i + 1) * nsub, 1)

        wait_scatter((nloop - 1) % 2)

        plsc.subcore_barrier()

        @pl.when(s < 15)
        def _():
            r0 = pl.multiple_of(s * sz_a, 8)
            pltpu.sync_copy(acc.at[pl.ds(r0, sz_a)],
                            out_hbm.at[c, pl.ds(r0, sz_a)])

        @pl.when(s == 15)
        def _():
            pltpu.sync_copy(acc.at[pl.ds(15 * sz_a, sz_b)],
                            out_hbm.at[c, pl.ds(15 * sz_a, sz_b)])

    return k(msg, fvec, row, zm)


def _edge_kernel(ga_ref, gb_ref, d_ref, ef_ref,
                 w1n_ref, w1ef_ref, b1_ref, w2_ref, b2_ref,
                 cnw1_ref, cnb1_ref, cnw2_ref, cnb2_ref,
                 msg_ref, fvec_ref):
    ga = ga_ref[...]
    gb = gb_ref[...]
    d = d_ref[...]
    rij = d[:, 0:3]
    vij = d[:, 3:6]
    nr = jnp.sqrt(jnp.sum(rij * rij, axis=1, keepdims=True))
    nv = jnp.sqrt(jnp.sum(vij * vij, axis=1, keepdims=True))
    w1n = w1n_ref[...]
    pre1 = (ga + gb + b1_ref[...]
            + nr * w1n[0:1, :] + nv * w1n[1:2, :]
            + jnp.dot(ef_ref[...], w1ef_ref[...],
                      preferred_element_type=jnp.float32))
    m1 = _silu(pre1)
    msg = _silu(jnp.dot(m1, w2_ref[...], preferred_element_type=jnp.float32)
                + b2_ref[...])
    msg_ref[...] = msg
    ch = _silu(jnp.dot(msg, cnw1_ref[...], preferred_element_type=jnp.float32)
               + cnb1_ref[...])
    cm = jnp.dot(ch, cnw2_ref[...], preferred_element_type=jnp.float32) \
        + cnb2_ref[0, 0]
    f = rij * cm
    ones = jnp.ones_like(cm)
    fvec_ref[...] = jnp.concatenate(
        [f, ones, jnp.zeros((f.shape[0], 12), jnp.float32)], axis=1)


def _edge_mlp(ga, gb, d, ef, w1n, w1ef, b1, w2, b2, cnw1, cnb1, cnw2, cnb2):
    e = ga.shape[0]
    te = 2000
    grid = (e // te,)
    full = lambda shp: pl.BlockSpec(shp, lambda i: (0, 0))
    de = ef.shape[1]
    return pl.pallas_call(
        _edge_kernel,
        grid=grid,
        in_specs=[
            pl.BlockSpec((te, H), lambda i: (i, 0)),
            pl.BlockSpec((te, H), lambda i: (i, 0)),
            pl.BlockSpec((te, 16), lambda i: (i, 0)),
            pl.BlockSpec((te, de), lambda i: (i, 0)),
            full((2, H)), full((de, H)), full((1, H)),
            full((H, H)), full((1, H)),
            full((H, H)), full((1, H)), full((H, 1)), full((1, 1)),
        ],
        out_specs=[
            pl.BlockSpec((te, H), lambda i: (i, 0)),
            pl.BlockSpec((te, 16), lambda i: (i, 0)),
        ],
        out_shape=[
            jax.ShapeDtypeStruct((e, H), jnp.float32),
            jax.ShapeDtypeStruct((e, 16), jnp.float32),
        ],
    )(ga, gb, d, ef, w1n, w1ef, b1, w2, b2, cnw1, cnb1, cnw2, cnb2)


def _fin_kernel(hpre_ref, accm0_ref,
                nnw1b_ref, nnb1_ref, nnw2_ref, nnb2_ref, hnew_ref):
    accm = accm0_ref[0]
    t = _silu(hpre_ref[...]
              + jnp.dot(accm, nnw1b_ref[...],
                        preferred_element_type=jnp.float32)
              + nnb1_ref[...])
    hnew_ref[...] = jnp.dot(t, nnw2_ref[...],
                            preferred_element_type=jnp.float32) + nnb2_ref[...]


def _finalize(hpre, accm2, nnw1b, nnb1, nnw2, nnb2):
    n = hpre.shape[0]
    tn = 2000
    full = lambda shp: pl.BlockSpec(shp, lambda i: (0, 0))
    return pl.pallas_call(
        _fin_kernel,
        grid=(n // tn,),
        in_specs=[
            pl.BlockSpec((tn, H), lambda i: (i, 0)),
            pl.BlockSpec((1, tn, H), lambda i: (0, i, 0)),
            full((H, H)), full((1, H)), full((H, H)), full((1, H)),
        ],
        out_specs=pl.BlockSpec((tn, H), lambda i: (i, 0)),
        out_shape=jax.ShapeDtypeStruct((n, H), jnp.float32),
    )(hpre, accm2, nnw1b, nnb1, nnw2, nnb2)


def kernel(x, h, edge_index, edge_fea, v,
           em_W1, em_b1, em_W2, em_b2,
           cn_W1, cn_b1, cn_W2, cn_b2,
           nn_W1, nn_b1, nn_W2, nn_b2,
           nx_W1, nx_b1, nx_W2, nx_b2,
           nv_W1, nv_b1, nv_W2, nv_b2):
    n = x.shape[0]
    row = edge_index[0]
    col = edge_index[1]

    w1n = em_W1[0:2]
    w1r = em_W1[2:2 + H]
    w1c = em_W1[2 + H:2 + 2 * H]
    w1ef = em_W1[2 + 2 * H:]
    nnw1t = nn_W1[:H]
    nnw1b = nn_W1[H:]

    ta, tb, hpre, gates = _precompute(
        h, w1r, w1c, nnw1t,
        nx_W1, nx_b1.reshape(1, H), nx_W2,
        nv_W1, nv_b1.reshape(1, H), nv_W2)
    gate_x = gates[:, 0:1] + nx_b2[0]
    gate_v = gates[:, 1:2] + nv_b2[0]

    xv = jnp.concatenate([x, v, jnp.zeros((n, DD - 22), jnp.float32)],
                         axis=1)
    ga, gb, d = _sc_gather(ta, tb, xv, row, col)

    msg, fvec = _edge_mlp(
        ga, gb, d, edge_fea, w1n, w1ef,
        em_b1.reshape(1, H), em_W2, em_b2.reshape(1, H),
        cn_W1, cn_b1.reshape(1, H), cn_W2, cn_b2.reshape(1, 1))

    zm = jnp.zeros((n, H), jnp.float32)
    acc2 = _sc_scatter(msg, fvec, row, zm)
    accf = acc2[1, :, :16]

    cnt = accf[:, 3:4]
    tot_f = jnp.clip(accf[:, 0:3] / jnp.clip(cnt, 1.0, None), -100.0, 100.0)

    v_new = gate_v * v + tot_f
    x_new = gate_x * x + tot_f

    h_new = _finalize(hpre, acc2, nnw1b,
                      nn_b1.reshape(1, H), nn_W2, nn_b2.reshape(1, H))
    return (x_new, v_new, h_new)


# speedup vs baseline: 6.4581x; 1.1511x over previous
import functools

import jax
import jax.numpy as jnp
from jax import lax
from jax.experimental import pallas as pl
from jax.experimental.pallas import tpu as pltpu
from jax.experimental.pallas import tpu_sc as plsc

H = 128
DD = 144
_NW = 32
_CH = 128


def _silu(u):
    return u * jax.nn.sigmoid(u)


def _pre_kernel(h_ref, w1r_ref, w1c_ref, nnw1t_ref,
                nxw1_ref, nxb1_ref, nxw2_ref,
                nvw1_ref, nvb1_ref, nvw2_ref,
                ta_ref, tb_ref, hpre_ref, g_ref):
    hh = h_ref[...]
    ta_ref[...] = jnp.dot(hh, w1r_ref[...], preferred_element_type=jnp.float32)
    tb_ref[...] = jnp.dot(hh, w1c_ref[...], preferred_element_type=jnp.float32)
    hpre_ref[...] = jnp.dot(hh, nnw1t_ref[...], preferred_element_type=jnp.float32)
    gx = _silu(jnp.dot(hh, nxw1_ref[...], preferred_element_type=jnp.float32)
               + nxb1_ref[...])
    gv = _silu(jnp.dot(hh, nvw1_ref[...], preferred_element_type=jnp.float32)
               + nvb1_ref[...])
    gxs = jnp.dot(gx, nxw2_ref[...], preferred_element_type=jnp.float32)
    gvs = jnp.dot(gv, nvw2_ref[...], preferred_element_type=jnp.float32)
    g_ref[...] = jnp.concatenate(
        [gxs, gvs, jnp.zeros_like(gxs), jnp.zeros_like(gvs)], axis=1)


def _precompute(h, w1r, w1c, nnw1t, nxw1, nxb1, nxw2, nvw1, nvb1, nvw2):
    n = h.shape[0]
    tn = 2000
    grid = (n // tn,)
    full = lambda shp: pl.BlockSpec(shp, lambda i: (0, 0))
    out = pl.pallas_call(
        _pre_kernel,
        grid=grid,
        in_specs=[
            pl.BlockSpec((tn, H), lambda i: (i, 0)),
            full((H, H)), full((H, H)), full((H, H)),
            full((H, H)), full((1, H)), full((H, 1)),
            full((H, H)), full((1, H)), full((H, 1)),
        ],
        out_specs=[
            pl.BlockSpec((tn, H), lambda i: (i, 0)),
            pl.BlockSpec((tn, H), lambda i: (i, 0)),
            pl.BlockSpec((tn, H), lambda i: (i, 0)),
            pl.BlockSpec((tn, 4), lambda i: (i, 0)),
        ],
        out_shape=[
            jax.ShapeDtypeStruct((n, H), jnp.float32),
            jax.ShapeDtypeStruct((n, H), jnp.float32),
            jax.ShapeDtypeStruct((n, H), jnp.float32),
            jax.ShapeDtypeStruct((n, 4), jnp.float32),
        ],
    )(h, w1r, w1c, nnw1t, nxw1, nxb1, nxw2, nvw1, nvb1, nvw2)
    return out


def _sc_gather(ta, tb, xv, row, col):
    n, dd = ta.shape
    e = row.shape[0]
    nchunk = e // _CH
    nloop = (nchunk + _NW - 1) // _NW
    mesh = plsc.VectorSubcoreMesh(core_axis_name="c", subcore_axis_name="s")

    @functools.partial(
        pl.kernel,
        out_type=[jax.ShapeDtypeStruct((e, dd), jnp.float32),
                  jax.ShapeDtypeStruct((e, dd), jnp.float32),
                  jax.ShapeDtypeStruct((e, 16), jnp.float32)],
        mesh=mesh,
        scratch_types=[
            pltpu.VMEM((_CH,), jnp.int32),
            pltpu.VMEM((_CH,), jnp.int32),
            pltpu.VMEM((_CH, dd), jnp.float32),
            pltpu.VMEM((_CH, dd), jnp.float32),
            pltpu.VMEM((_CH, dd), jnp.float32),
            pltpu.VMEM((_CH, dd), jnp.float32),
            pltpu.VMEM((_CH, dd), jnp.float32),
            pltpu.VMEM((_CH, dd), jnp.float32),
            pltpu.VMEM((_CH, 16), jnp.float32),
            pltpu.SemaphoreType.DMA,
            pltpu.SemaphoreType.DMA,
            pltpu.SemaphoreType.DMA,
            pltpu.SemaphoreType.DMA,
        ],
    )
    def k(ta_hbm, tb_hbm, xv_hbm, row_hbm, col_hbm, ga_hbm, gb_hbm, d_hbm,
          rowi, coli, abuf0, abuf1, bbuf0, bbuf1, xr, xc, dbuf,
          sema, semb, semc, semd):
        wid = lax.axis_index("s") * 2 + lax.axis_index("c")
        ab = (abuf0, abuf1)
        bb = (bbuf0, bbuf1)

        def load_idx(c):
            base = c * _CH
            pltpu.sync_copy(row_hbm.at[pl.ds(base, _CH)], rowi)
            pltpu.sync_copy(col_hbm.at[pl.ds(base, _CH)], coli)

        def issue(p):
            pltpu.async_copy(ta_hbm.at[rowi], ab[p], sema)
            pltpu.async_copy(tb_hbm.at[coli], bb[p], semb)
            pltpu.async_copy(xv_hbm.at[rowi], xr, semc)
            pltpu.async_copy(xv_hbm.at[coli], xc, semd)

        def wait(p):
            pltpu.make_async_copy(ta_hbm.at[rowi], ab[p], sema).wait()
            pltpu.make_async_copy(tb_hbm.at[coli], bb[p], semb).wait()
            pltpu.make_async_copy(xv_hbm.at[rowi], xr, semc).wait()
            pltpu.make_async_copy(xv_hbm.at[coli], xc, semd).wait()

        def step(c, p):
            @pl.when(c < nchunk)
            def _():
                wait(p)
                for r in range(_CH):
                    dbuf[r, :] = xr[r, pl.ds(0, 16)] - xc[r, pl.ds(0, 16)]

                @pl.when(c + _NW < nchunk)
                def _():
                    load_idx(c + _NW)
                    issue(1 - p)

                base = c * _CH
                pltpu.sync_copy(ab[p], ga_hbm.at[pl.ds(base, _CH)])
                pltpu.sync_copy(bb[p], gb_hbm.at[pl.ds(base, _CH)])
                pltpu.sync_copy(dbuf, d_hbm.at[pl.ds(base, _CH)])

        load_idx(wid)
        issue(0)

        @pl.loop(0, (nloop + 1) // 2)
        def _(i):
            step(wid + (2 * i) * _NW, 0)
            step(wid + (2 * i + 1) * _NW, 1)

    return k(ta, tb, xv, row, col)


def _sc_scatter(msg, fvec, row, zm):
    e, hh = msg.shape
    n = zm.shape[0]
    chs = 64
    nchunk = e // chs
    nsub = 16
    nloop = (nchunk + nsub - 1) // nsub
    mesh = plsc.VectorSubcoreMesh(core_axis_name="c", subcore_axis_name="s")

    @functools.partial(
        pl.kernel,
        out_type=jax.ShapeDtypeStruct((2, n, hh), jnp.float32),
        mesh=mesh,
        scratch_types=[
            pltpu.VMEM((chs,), jnp.int32),
            pltpu.VMEM((chs,), jnp.int32),
            pltpu.VMEM((chs, hh), jnp.float32),
            pltpu.VMEM((chs, hh), jnp.float32),
            pltpu.VMEM((chs, 16), jnp.float32),
            pltpu.VMEM((chs, 16), jnp.float32),
            pltpu.VMEM_SHARED((n, hh), jnp.float32),
            pltpu.SemaphoreType.DMA,
            pltpu.SemaphoreType.DMA,
            pltpu.SemaphoreType.DMA,
        ],
    )
    def k(msg_hbm, fvec_hbm, row_hbm, zm_hbm, out_hbm,
          rowi0, rowi1, mbuf0, mbuf1, fbuf0, fbuf1, acc,
          semr, semm, sems):
        c = lax.axis_index("c")
        s = lax.axis_index("s")
        ri = (rowi0, rowi1)
        mb = (mbuf0, mbuf1)
        fb = (fbuf0, fbuf1)
        sz_a, sz_b = 624, n - 15 * 624

        @pl.when(s < 15)
        def _():
            r0 = pl.multiple_of(s * sz_a, 8)
            pltpu.sync_copy(zm_hbm.at[pl.ds(r0, sz_a)], acc.at[pl.ds(r0, sz_a)])

        @pl.when(s == 15)
        def _():
            pltpu.sync_copy(zm_hbm.at[pl.ds(15 * sz_a, sz_b)],
                            acc.at[pl.ds(15 * sz_a, sz_b)])

        @pl.when(c == 1)
        def _():
            pltpu.sync_copy(zm_hbm.at[pl.ds(0, chs)], mbuf0)
            pltpu.sync_copy(zm_hbm.at[pl.ds(0, chs)], mbuf1)

        plsc.subcore_barrier()

        def issue_loads(j, p):
            base = j * chs
            pltpu.async_copy(row_hbm.at[pl.ds(base, chs)], ri[p], semr)

            @pl.when(c == 0)
            def _():
                pltpu.async_copy(msg_hbm.at[pl.ds(base, chs)], mb[p], semm)

            @pl.when(c == 1)
            def _():
                pltpu.async_copy(fvec_hbm.at[pl.ds(base, chs)], fb[p], semm)

        def wait_loads(p):
            pltpu.make_async_copy(row_hbm.at[pl.ds(0, chs)], ri[p],
                                  semr).wait()

            @pl.when(c == 0)
            def _():
                pltpu.make_async_copy(msg_hbm.at[pl.ds(0, chs)], mb[p],
                                      semm).wait()

            @pl.when(c == 1)
            def _():
                pltpu.make_async_copy(fvec_hbm.at[pl.ds(0, chs)], fb[p],
                                      semm).wait()

        def wait_scatter(p):
            pltpu.make_async_copy(mb[p], acc.at[ri[p]], sems).wait()

        def step(j, p):
            @pl.when(j < nchunk)
            def _():
                wait_loads(p)

                @pl.when(j > s)
                def _():
                    wait_scatter(1 - p)

                @pl.when(j + nsub < nchunk)
                def _():
                    issue_loads(j + nsub, 1 - p)

                @pl.when(c == 1)
                def _():
                    for r in range(chs):
                        mb[p][r, pl.ds(0, 16)] = fb[p][r, :]

                pltpu.async_copy(mb[p], acc.at[ri[p]], sems, add=True)

        issue_loads(s, 0)

        @pl.loop(0, (nloop + 1) // 2)
        def _(i):
            step(s + (2 * i) * nsub, 0)
            step(s + (2 * i + 1) * nsub, 1)

        wait_scatter((nloop - 1) % 2)

        plsc.subcore_barrier()

        @pl.when(s < 15)
        def _():
            r0 = pl.multiple_of(s * sz_a, 8)
            pltpu.sync_copy(acc.at[pl.ds(r0, sz_a)],
                            out_hbm.at[c, pl.ds(r0, sz_a)])

        @pl.when(s == 15)
        def _():
            pltpu.sync_copy(acc.at[pl.ds(15 * sz_a, sz_b)],
                            out_hbm.at[c, pl.ds(15 * sz_a, sz_b)])

    return k(msg, fvec, row, zm)


def _edge_kernel(ga_ref, gb_ref, d_ref, ef_ref,
                 w1n_ref, w1ef_ref, b1_ref, w2_ref, b2_ref,
                 cnw1_ref, cnb1_ref, cnw2_ref, cnb2_ref,
                 msg_ref, fvec_ref):
    ga = ga_ref[...]
    gb = gb_ref[...]
    d = d_ref[...]
    rij = d[:, 0:3]
    vij = d[:, 3:6]
    nr = jnp.sqrt(jnp.sum(rij * rij, axis=1, keepdims=True))
    nv = jnp.sqrt(jnp.sum(vij * vij, axis=1, keepdims=True))
    w1n = w1n_ref[...]
    pre1 = (ga + gb + b1_ref[...]
            + nr * w1n[0:1, :] + nv * w1n[1:2, :]
            + jnp.dot(ef_ref[...], w1ef_ref[...],
                      preferred_element_type=jnp.float32))
    m1 = _silu(pre1)
    msg = _silu(jnp.dot(m1, w2_ref[...], preferred_element_type=jnp.float32)
                + b2_ref[...])
    msg_ref[...] = msg
    ch = _silu(jnp.dot(msg, cnw1_ref[...], preferred_element_type=jnp.float32)
               + cnb1_ref[...])
    cm = jnp.dot(ch, cnw2_ref[...], preferred_element_type=jnp.float32) \
        + cnb2_ref[0, 0]
    f = rij * cm
    ones = jnp.ones_like(cm)
    fvec_ref[...] = jnp.concatenate(
        [f, ones, jnp.zeros((f.shape[0], 12), jnp.float32)], axis=1)


def _edge_mlp(ga, gb, d, ef, w1n, w1ef, b1, w2, b2, cnw1, cnb1, cnw2, cnb2):
    e = ga.shape[0]
    te = 2000
    grid = (e // te,)
    full = lambda shp: pl.BlockSpec(shp, lambda i: (0, 0))
    de = ef.shape[1]
    return pl.pallas_call(
        _edge_kernel,
        grid=grid,
        in_specs=[
            pl.BlockSpec((te, H), lambda i: (i, 0)),
            pl.BlockSpec((te, H), lambda i: (i, 0)),
            pl.BlockSpec((te, 16), lambda i: (i, 0)),
            pl.BlockSpec((te, de), lambda i: (i, 0)),
            full((2, H)), full((de, H)), full((1, H)),
            full((H, H)), full((1, H)),
            full((H, H)), full((1, H)), full((H, 1)), full((1, 1)),
        ],
        out_specs=[
            pl.BlockSpec((te, H), lambda i: (i, 0)),
            pl.BlockSpec((te, 16), lambda i: (i, 0)),
        ],
        out_shape=[
            jax.ShapeDtypeStruct((e, H), jnp.float32),
            jax.ShapeDtypeStruct((e, 16), jnp.float32),
        ],
    )(ga, gb, d, ef, w1n, w1ef, b1, w2, b2, cnw1, cnb1, cnw2, cnb2)


def _fin_kernel(hpre_ref, accm0_ref, accm1_ref,
                nnw1b_ref, nnb1_ref, nnw2_ref, nnb2_ref, hnew_ref):
    accm = accm0_ref[0] + accm1_ref[0]
    t = _silu(hpre_ref[...]
              + jnp.dot(accm, nnw1b_ref[...],
                        preferred_element_type=jnp.float32)
              + nnb1_ref[...])
    hnew_ref[...] = jnp.dot(t, nnw2_ref[...],
                            preferred_element_type=jnp.float32) + nnb2_ref[...]


def _finalize(hpre, acca, accb, nnw1b, nnb1, nnw2, nnb2):
    n = hpre.shape[0]
    tn = 2000
    full = lambda shp: pl.BlockSpec(shp, lambda i: (0, 0))
    return pl.pallas_call(
        _fin_kernel,
        grid=(n // tn,),
        in_specs=[
            pl.BlockSpec((tn, H), lambda i: (i, 0)),
            pl.BlockSpec((1, tn, H), lambda i: (0, i, 0)),
            pl.BlockSpec((1, tn, H), lambda i: (0, i, 0)),
            full((H, H)), full((1, H)), full((H, H)), full((1, H)),
        ],
        out_specs=pl.BlockSpec((tn, H), lambda i: (i, 0)),
        out_shape=jax.ShapeDtypeStruct((n, H), jnp.float32),
    )(hpre, acca, accb, nnw1b, nnb1, nnw2, nnb2)


def kernel(x, h, edge_index, edge_fea, v,
           em_W1, em_b1, em_W2, em_b2,
           cn_W1, cn_b1, cn_W2, cn_b2,
           nn_W1, nn_b1, nn_W2, nn_b2,
           nx_W1, nx_b1, nx_W2, nx_b2,
           nv_W1, nv_b1, nv_W2, nv_b2):
    n = x.shape[0]
    row = edge_index[0]
    col = edge_index[1]

    w1n = em_W1[0:2]
    w1r = em_W1[2:2 + H]
    w1c = em_W1[2 + H:2 + 2 * H]
    w1ef = em_W1[2 + 2 * H:]
    nnw1t = nn_W1[:H]
    nnw1b = nn_W1[H:]

    ta, tb, hpre, gates = _precompute(
        h, w1r, w1c, nnw1t,
        nx_W1, nx_b1.reshape(1, H), nx_W2,
        nv_W1, nv_b1.reshape(1, H), nv_W2)
    gate_x = gates[:, 0:1] + nx_b2[0]
    gate_v = gates[:, 1:2] + nv_b2[0]

    xv = jnp.concatenate([x, v, jnp.zeros((n, DD - 22), jnp.float32)],
                         axis=1)
    eh = row.shape[0] // 2
    zm = jnp.zeros((n, H), jnp.float32)
    halves = []
    for lo, hi in ((0, eh), (eh, 2 * eh)):
        ga, gb, d = _sc_gather(ta, tb, xv, row[lo:hi], col[lo:hi])
        msg, fvec = _edge_mlp(
            ga, gb, d, edge_fea[lo:hi], w1n, w1ef,
            em_b1.reshape(1, H), em_W2, em_b2.reshape(1, H),
            cn_W1, cn_b1.reshape(1, H), cn_W2, cn_b2.reshape(1, 1))
        halves.append(_sc_scatter(msg, fvec, row[lo:hi], zm))
    acca, accb = halves
    accf = acca[1, :, :16] + accb[1, :, :16]

    cnt = accf[:, 3:4]
    tot_f = jnp.clip(accf[:, 0:3] / jnp.clip(cnt, 1.0, None), -100.0, 100.0)

    v_new = gate_v * v + tot_f
    x_new = gate_x * x + tot_f

    h_new = _finalize(hpre, acca, accb, nnw1b,
                      nn_b1.reshape(1, H), nn_W2, nn_b2.reshape(1, H))
    return (x_new, v_new, h_new)
